# Initial kernel scaffold; baseline (speedup 1.0000x reference)
import jax
import jax.numpy as jnp
from jax.experimental import pallas as pl


def kernel(num_nodes, num_graphs, atomic_numbers, edge_list, edge_lengths, edge_vectors, node_coordinates, node_graph_index, emb_scalar, emb_tri, rbf_W, rbf_b, phi_W1, phi_b1, phi_W2, phi_b2, U_W, V_W, Z1w, Z2w, gpZ1_W, gpZ2_W, upd_W1, upd_b1, upd_W2, upd_b2):
    raise NotImplementedError("write your pallas kernel here")


# speedup vs baseline: 10.3570x; 10.3570x over previous
import functools
import math

import jax
import jax.numpy as jnp
from jax import lax
from jax.experimental import pallas as pl
from jax.experimental.pallas import tpu as pltpu
from jax.experimental.pallas import tpu_sc as plsc

N = 10000
E = 160000
D = 128
G = 64
NUM_RBF = 20
R_CUT = 5.0
NUM_TYPES = 5

_GP_IDX = ((0, 1, 2, 3, 4, 5, 6, 7), (1, 0, 4, 14, 2, 7, 11, 5), (2, 12, 0, 5, 9, 3, 7, 6), (3, 6, 13, 0, 7, 10, 1, 4), (4, 10, 1, 7, 8, 14, 5, 11), (5, 7, 11, 2, 6, 8, 12, 9), (6, 3, 7, 9, 13, 4, 8, 10), (7, 5, 6, 4, 11, 9, 10, 8))
_W_IDX = ((0, 1, 1, 1, 2, 2, 2, 3), (4, 5, 6, 6, 7, 8, 7, 9), (4, 6, 5, 6, 7, 7, 8, 9), (4, 6, 6, 5, 8, 7, 7, 9), (10, 11, 11, 12, 13, 14, 14, 15), (10, 12, 11, 11, 14, 13, 14, 15), (10, 11, 12, 11, 14, 14, 13, 15), (16, 17, 17, 17, 18, 18, 18, 19))

NB = 400
NBLK = N // NB
EB = 2000
RBT_W = 656
MSRC_W = 1536

NCHUNK = 10
CH = 1024
NPAD = NCHUNK * CH
ACC_ROWS = 1088
DUMMY = 1087
EPT = E // 16
STRIP = 400
NSTRIP = EPT // STRIP
K = 16


def _silu(x):
    return x / (1.0 + jnp.exp(-x))


def _init_body(an_ref, es_ref, et_ref, out_ref):
    an = an_ref[...]
    tt = lax.broadcasted_iota(jnp.int32, (1, NUM_TYPES), 1)
    oh = (an == tt).astype(jnp.float32)
    s0 = jnp.dot(oh, es_ref[...], preferred_element_type=jnp.float32)
    s7 = jnp.dot(oh, et_ref[...], preferred_element_type=jnp.float32)
    out_ref[:, 0:D] = s0
    out_ref[:, D:7 * D] = jnp.zeros((NB, 6 * D), jnp.float32)
    out_ref[:, 7 * D:8 * D] = s7


def _init_state(an2, emb_s, emb_t):
    return pl.pallas_call(
        _init_body,
        grid=(NBLK,),
        in_specs=[
            pl.BlockSpec((NB, 1), lambda i: (i, 0)),
            pl.BlockSpec((NUM_TYPES, D), lambda i: (0, 0)),
            pl.BlockSpec((NUM_TYPES, D), lambda i: (0, 0)),
        ],
        out_specs=pl.BlockSpec((NB, 8 * D), lambda i: (i, 0)),
        out_shape=jax.ShapeDtypeStruct((N, 8 * D), jnp.float32),
    )(an2, emb_s, emb_t)


def _rbt_body(el_ref, ev_ref, w_ref, b_ref, out_ref):
    el = el_ref[...]
    r = jnp.maximum(el, 1e-6)
    k = lax.broadcasted_iota(jnp.int32, (1, NUM_RBF), 1).astype(jnp.float32) + 1.0
    freqs = k * (math.pi / R_CUT)
    rbf = jnp.sin(r * freqs) / r
    cutoff = 0.5 * (jnp.cos((math.pi / R_CUT) * el) + 1.0)
    cutoff = cutoff * (el < R_CUT).astype(jnp.float32)
    out640 = (jnp.dot(rbf, w_ref[...], preferred_element_type=jnp.float32)
              + b_ref[...]) * cutoff
    out_ref[:, 0:5 * D] = out640
    out_ref[:, 5 * D:5 * D + 3] = ev_ref[...]
    out_ref[:, 5 * D + 3:RBT_W] = jnp.zeros((EB, RBT_W - 5 * D - 3), jnp.float32)


def _rbt(el2, ev, w, b2):
    return pl.pallas_call(
        _rbt_body,
        grid=(E // EB,),
        in_specs=[
            pl.BlockSpec((EB, 1), lambda i: (i, 0)),
            pl.BlockSpec((EB, 3), lambda i: (i, 0)),
            pl.BlockSpec((NUM_RBF, 5 * D), lambda i: (0, 0)),
            pl.BlockSpec((1, 5 * D), lambda i: (0, 0)),
        ],
        out_specs=pl.BlockSpec((EB, RBT_W), lambda i: (i, 0)),
        out_shape=jax.ShapeDtypeStruct((E, RBT_W), jnp.float32),
    )(el2, ev, w, b2)


def _msrc_body(st_ref, w1_ref, b1_ref, w2_ref, b2_ref, out_ref):
    s0 = st_ref[:, 0:D]
    h = _silu(jnp.dot(s0, w1_ref[...], preferred_element_type=jnp.float32)
              + b1_ref[...])
    phi = jnp.dot(h, w2_ref[...], preferred_element_type=jnp.float32) + b2_ref[...]
    out_ref[:, 0:5 * D] = phi
    out_ref[:, 5 * D:MSRC_W] = st_ref[:, D:8 * D]


def _msrc(state, w1, b1, w2, b2):
    return pl.pallas_call(
        _msrc_body,
        grid=(NBLK,),
        in_specs=[
            pl.BlockSpec((NB, 8 * D), lambda i: (i, 0)),
            pl.BlockSpec((D, 5 * D), lambda i: (0, 0)),
            pl.BlockSpec((1, 5 * D), lambda i: (0, 0)),
            pl.BlockSpec((5 * D, 5 * D), lambda i: (0, 0)),
            pl.BlockSpec((1, 5 * D), lambda i: (0, 0)),
        ],
        out_specs=pl.BlockSpec((NB, MSRC_W), lambda i: (i, 0)),
        out_shape=jax.ShapeDtypeStruct((N, MSRC_W), jnp.float32),
    )(state, w1, b1, w2, b2)


def _edge_body(snd_hbm, rcv_hbm, msrc_hbm, rbt_hbm, agg_hbm,
               acc, sstrip, rstrip, feid, fsnd, frcv,
               eidx, sidx, ridx, mrows, rrows, msg,
               sem0, sem1, sem2):
    core = lax.axis_index("c")
    sub = lax.axis_index("s")
    tile_ebase = sub * EPT
    lanes = lax.iota(jnp.int32, 16)

    def edge_compute(e, _):
        evv = rrows[e, pl.ds(5 * D, 16)]
        ev0 = evv[0]
        ev1 = evv[1]
        ev2 = evv[2]
        for j in range(8):
            o = j * 16
            g_s = mrows[e, pl.ds(o, 16)] * rrows[e, pl.ds(o, 16)]
            msg[e, pl.ds(o, 16)] = g_s
            g_v = mrows[e, pl.ds(D + o, 16)] * rrows[e, pl.ds(D + o, 16)]
            g_d = mrows[e, pl.ds(2 * D + o, 16)] * rrows[e, pl.ds(2 * D + o, 16)]
            sv0 = mrows[e, pl.ds(5 * D + o, 16)]
            sv1 = mrows[e, pl.ds(6 * D + o, 16)]
            sv2 = mrows[e, pl.ds(7 * D + o, 16)]
            msg[e, pl.ds(D + o, 16)] = g_v * sv0 + g_d * ev0
            msg[e, pl.ds(2 * D + o, 16)] = g_v * sv1 + g_d * ev1
            msg[e, pl.ds(3 * D + o, 16)] = g_v * sv2 + g_d * ev2
            g_b = mrows[e, pl.ds(3 * D + o, 16)] * rrows[e, pl.ds(3 * D + o, 16)]
            msg[e, pl.ds(4 * D + o, 16)] = g_b * mrows[e, pl.ds(8 * D + o, 16)]
            msg[e, pl.ds(5 * D + o, 16)] = g_b * mrows[e, pl.ds(9 * D + o, 16)]
            msg[e, pl.ds(6 * D + o, 16)] = g_b * mrows[e, pl.ds(10 * D + o, 16)]
            g_t = mrows[e, pl.ds(4 * D + o, 16)] * rrows[e, pl.ds(4 * D + o, 16)]
            msg[e, pl.ds(7 * D + o, 16)] = g_t * mrows[e, pl.ds(11 * D + o, 16)]
        return 0

    def process_batch(nvalid):
        valid = lanes < nvalid
        eidx[...] = jnp.where(valid, feid[pl.ds(0, 16)], 0)
        sidx[...] = jnp.where(valid, fsnd[pl.ds(0, 16)], 0)
        ridx[...] = jnp.where(valid, frcv[pl.ds(0, 16)], DUMMY)
        cp_m = pltpu.async_copy(msrc_hbm.at[sidx], mrows, sem1)
        cp_r = pltpu.async_copy(rbt_hbm.at[eidx], rrows, sem2)
        cp_m.wait()
        cp_r.wait()
        lax.fori_loop(0, K, edge_compute, 0)
        pltpu.sync_copy(msg, acc.at[ridx], add=True)

    def scan_edges(lo, hi):
        def strip_body(s, fill):
            sbase = tile_ebase + s * STRIP
            pltpu.sync_copy(snd_hbm.at[pl.ds(sbase, STRIP)], sstrip)
            pltpu.sync_copy(rcv_hbm.at[pl.ds(sbase, STRIP)], rstrip)

            def vbody(j, fill):
                rv = rstrip[pl.ds(j * 16, 16)]
                sv = sstrip[pl.ds(j * 16, 16)]
                m = (rv >= lo) & (rv < hi)
                mi = m.astype(jnp.int32)
                pos = fill + plsc.cumsum(mi) - 1
                eidv = sbase + j * 16 + lanes
                plsc.store_scatter(feid, [pos], eidv, mask=m)
                plsc.store_scatter(fsnd, [pos], sv, mask=m)
                plsc.store_scatter(frcv, [pos], rv - lo, mask=m)
                fill = fill + jnp.sum(mi)

                @pl.when(fill >= K)
                def _flush():
                    process_batch(K)
                    feid[pl.ds(0, 16)] = feid[pl.ds(16, 16)]
                    fsnd[pl.ds(0, 16)] = fsnd[pl.ds(16, 16)]
                    frcv[pl.ds(0, 16)] = frcv[pl.ds(16, 16)]
                return jnp.where(fill >= K, fill - K, fill)
            return lax.fori_loop(0, STRIP // 16, vbody, fill)
        fill = lax.fori_loop(0, NSTRIP, strip_body, 0)

        @pl.when(fill > 0)
        def _drain():
            process_batch(fill)

    def chunk_body(chunk, _):
        lo = chunk * CH

        @pl.when(core == (chunk % 2))
        def _process():
            def zb(e, _):
                for j in range(64):
                    msg[e, pl.ds(j * 16, 16)] = jnp.zeros((16,), jnp.float32)
                return 0
            lax.fori_loop(0, K, zb, 0)
            zoff = jnp.where(sub < 8, sub * 72, 576 + (sub - 8) * 64)
            for q in range(4):
                pltpu.sync_copy(msg, acc.at[pl.ds(zoff + q * K, K)])

            @pl.when(sub < 8)
            def _zextra():
                pltpu.sync_copy(msg.at[pl.ds(0, 8)],
                                acc.at[pl.ds(sub * 72 + 4 * K, 8)])
            plsc.subcore_barrier()
            scan_edges(lo, lo + CH)
            plsc.subcore_barrier()

            @pl.when(sub < 8)
            def _writeback():
                pltpu.sync_copy(acc.at[pl.ds(sub * (CH // 8), CH // 8)],
                                agg_hbm.at[pl.ds(lo + sub * (CH // 8), CH // 8)])
            plsc.subcore_barrier()
        return 0

    lax.fori_loop(0, NCHUNK, chunk_body, 0)


def _edge_aggregate(senders, receivers, msrc, rbt):
    mesh = plsc.VectorSubcoreMesh(core_axis_name="c", subcore_axis_name="s")
    f = pl.kernel(
        _edge_body,
        mesh=mesh,
        compiler_params=pltpu.CompilerParams(use_tc_tiling_on_sc=False, needs_layout_passes=False),
        out_type=jax.ShapeDtypeStruct((NPAD, 8 * D), jnp.float32),
        scratch_types=[
            pltpu.VMEM_SHARED((ACC_ROWS, 8 * D), jnp.float32),
            pltpu.VMEM((STRIP,), jnp.int32),
            pltpu.VMEM((STRIP,), jnp.int32),
            pltpu.VMEM((48,), jnp.int32),
            pltpu.VMEM((48,), jnp.int32),
            pltpu.VMEM((48,), jnp.int32),
            pltpu.VMEM((16,), jnp.int32),
            pltpu.VMEM((16,), jnp.int32),
            pltpu.VMEM((16,), jnp.int32),
            pltpu.VMEM((K, MSRC_W), jnp.float32),
            pltpu.VMEM((K, RBT_W), jnp.float32),
            pltpu.VMEM((K, 8 * D), jnp.float32),
            pltpu.SemaphoreType.DMA,
            pltpu.SemaphoreType.DMA,
            pltpu.SemaphoreType.DMA,
        ],
    )
    return f(senders, receivers, msrc, rbt)


def _node_body(st_ref, agg_ref, an_ref, uw_ref, vw_ref, z1w_ref, z2w_ref,
               g1_ref, g2_ref, w1_ref, b1_ref, w2_ref, b2_ref, out_ref):
    st = [st_ref[:, c * D:(c + 1) * D] + agg_ref[:, c * D:(c + 1) * D]
          for c in range(8)]
    uw = uw_ref[...]
    vw = vw_ref[...]
    U = [jnp.dot(st[c], uw, preferred_element_type=jnp.float32) for c in range(8)]
    V = [jnp.dot(st[c], vw, preferred_element_type=jnp.float32) for c in range(8)]

    def wmp(A, B, w_ref):
        out = [None] * 8
        for i in range(8):
            for j in range(8):
                gp = _GP_IDX[i][j]
                w = w_ref[_W_IDX[i][j]:_W_IDX[i][j] + 1, :]
                term = A[i] * B[j] * w
                c = gp if gp < 8 else gp - 8
                if out[c] is None:
                    out[c] = term if gp < 8 else -term
                else:
                    out[c] = out[c] + term if gp < 8 else out[c] - term
        return out

    Z1 = wmp(U, V, z1w_ref)
    g1 = g1_ref[...]
    Z1l = [jnp.dot(Z1[c], g1, preferred_element_type=jnp.float32) for c in range(8)]
    Z2 = wmp(U, Z1l, z2w_ref)
    g2 = g2_ref[...]
    Z2l = [jnp.dot(Z2[c], g2, preferred_element_type=jnp.float32) for c in range(8)]

    v_norm = jnp.sqrt(V[1] * V[1] + V[2] * V[2] + V[3] * V[3])
    upd_in = jnp.concatenate([st[0], v_norm], axis=1)
    an = an_ref[...]
    a = jnp.zeros((NB, 4 * D), jnp.float32)
    for t in range(NUM_TYPES):
        h1 = _silu(jnp.dot(upd_in, w1_ref[t], preferred_element_type=jnp.float32)
                   + b1_ref[t])
        out_t = jnp.dot(h1, w2_ref[t], preferred_element_type=jnp.float32) + b2_ref[t]
        a = jnp.where(an == t, out_t, a)
    ach = [a[:, q * D:(q + 1) * D] for q in range(4)]
    new = [None] * 8
    new[0] = st[0] + ach[0] * (U[0] + Z1l[0] + Z2l[0])
    for i in range(3):
        new[1 + i] = st[1 + i] + ach[1] * (U[1 + i] + Z1l[1 + i] + Z2l[1 + i])
        new[4 + i] = st[4 + i] + ach[2] * (U[4 + i] + Z1l[4 + i] + Z2l[4 + i])
    new[7] = st[7] + ach[3] * (U[7] + Z1l[7] + Z2l[7])
    out_ref[...] = jnp.concatenate(new, axis=1)


def _node_update(state, agg, an2, uw, vw, z1w, z2w, g1, g2, w1, b1, w2, b2):
    return pl.pallas_call(
        _node_body,
        grid=(NBLK,),
        in_specs=[
            pl.BlockSpec((NB, 8 * D), lambda i: (i, 0)),
            pl.BlockSpec((NB, 8 * D), lambda i: (i, 0)),
            pl.BlockSpec((NB, 1), lambda i: (i, 0)),
            pl.BlockSpec((D, D), lambda i: (0, 0)),
            pl.BlockSpec((D, D), lambda i: (0, 0)),
            pl.BlockSpec((NUM_RBF, D), lambda i: (0, 0)),
            pl.BlockSpec((NUM_RBF, D), lambda i: (0, 0)),
            pl.BlockSpec((D, D), lambda i: (0, 0)),
            pl.BlockSpec((D, D), lambda i: (0, 0)),
            pl.BlockSpec((NUM_TYPES, 2 * D, 4 * D), lambda i: (0, 0, 0)),
            pl.BlockSpec((NUM_TYPES, 1, 4 * D), lambda i: (0, 0, 0)),
            pl.BlockSpec((NUM_TYPES, 4 * D, 4 * D), lambda i: (0, 0, 0)),
            pl.BlockSpec((NUM_TYPES, 1, 4 * D), lambda i: (0, 0, 0)),
        ],
        out_specs=pl.BlockSpec((NB, 8 * D), lambda i: (i, 0)),
        out_shape=jax.ShapeDtypeStruct((N, 8 * D), jnp.float32),
    )(state, agg, an2, uw, vw, z1w, z2w, g1, g2, w1, b1, w2, b2)


def _readout_body(st_ref, co_ref, ngi_ref, out_ref, acc):
    i = pl.program_id(0)

    @pl.when(i == 0)
    def _init():
        acc[...] = jnp.zeros((G, D), jnp.float32)

    st0 = st_ref[:, 0:D]
    q = jnp.sum(st0, axis=1, keepdims=True)
    cols = []
    for ax in range(3):
        m = jnp.sum(st_ref[:, (1 + ax) * D:(2 + ax) * D], axis=1, keepdims=True)
        cols.append(m + q * co_ref[:, ax:ax + 1])
    mu_pad = jnp.concatenate(cols + [jnp.zeros((NB, D - 3), jnp.float32)], axis=1)
    ngi = ngi_ref[0]
    gcol = lax.broadcasted_iota(jnp.int32, (G, 1), 0)
    ohT = (gcol == ngi).astype(jnp.float32)
    acc[...] = acc[...] + jnp.dot(ohT, mu_pad, preferred_element_type=jnp.float32)

    @pl.when(i == NBLK - 1)
    def _fin():
        d3 = acc[:, 0:3]
        out_ref[...] = jnp.sqrt(jnp.sum(d3 * d3, axis=1, keepdims=True))


def _readout(state, coords, ngi3):
    return pl.pallas_call(
        _readout_body,
        grid=(NBLK,),
        in_specs=[
            pl.BlockSpec((NB, 8 * D), lambda i: (i, 0)),
            pl.BlockSpec((NB, 3), lambda i: (i, 0)),
            pl.BlockSpec((1, 1, NB), lambda i: (i, 0, 0)),
        ],
        out_specs=pl.BlockSpec((G, 1), lambda i: (0, 0)),
        out_shape=jax.ShapeDtypeStruct((G, 1), jnp.float32),
        scratch_shapes=[pltpu.VMEM((G, D), jnp.float32)],
    )(state, coords, ngi3)


def kernel(num_nodes, num_graphs, atomic_numbers, edge_list, edge_lengths,
           edge_vectors, node_coordinates, node_graph_index, emb_scalar,
           emb_tri, rbf_W, rbf_b, phi_W1, phi_b1, phi_W2, phi_b2, U_W, V_W,
           Z1w, Z2w, gpZ1_W, gpZ2_W, upd_W1, upd_b1, upd_W2, upd_b2):
    an2 = atomic_numbers.reshape(N, 1)
    ngi3 = node_graph_index.reshape(NBLK, 1, NB)
    senders = edge_list[:, 0]
    receivers = edge_list[:, 1]
    el2 = edge_lengths.reshape(E, 1)

    state = _init_state(an2, emb_scalar, emb_tri)
    rbts = [_rbt(el2, edge_vectors, rbf_W[rr], rbf_b[rr].reshape(1, 5 * D))
            for rr in range(2)]
    for rr in range(2):
        msrc = _msrc(state, phi_W1[rr], phi_b1[rr].reshape(1, 5 * D),
                     phi_W2[rr], phi_b2[rr].reshape(1, 5 * D))
        agg = _edge_aggregate(senders, receivers, msrc, rbts[rr])
        state = _node_update(
            state, agg, an2, U_W[rr], V_W[rr], Z1w[rr], Z2w[rr],
            gpZ1_W[rr], gpZ2_W[rr], upd_W1[rr],
            upd_b1[rr].reshape(NUM_TYPES, 1, 4 * D), upd_W2[rr],
            upd_b2[rr].reshape(NUM_TYPES, 1, 4 * D))
    return _readout(state, node_coordinates, ngi3)


# speedup vs baseline: 11.3855x; 1.0993x over previous
import functools
import math

import jax
import jax.numpy as jnp
from jax import lax
from jax.experimental import pallas as pl
from jax.experimental.pallas import tpu as pltpu
from jax.experimental.pallas import tpu_sc as plsc

N = 10000
E = 160000
D = 128
G = 64
NUM_RBF = 20
R_CUT = 5.0
NUM_TYPES = 5

_GP_IDX = ((0, 1, 2, 3, 4, 5, 6, 7), (1, 0, 4, 14, 2, 7, 11, 5), (2, 12, 0, 5, 9, 3, 7, 6), (3, 6, 13, 0, 7, 10, 1, 4), (4, 10, 1, 7, 8, 14, 5, 11), (5, 7, 11, 2, 6, 8, 12, 9), (6, 3, 7, 9, 13, 4, 8, 10), (7, 5, 6, 4, 11, 9, 10, 8))
_W_IDX = ((0, 1, 1, 1, 2, 2, 2, 3), (4, 5, 6, 6, 7, 8, 7, 9), (4, 6, 5, 6, 7, 7, 8, 9), (4, 6, 6, 5, 8, 7, 7, 9), (10, 11, 11, 12, 13, 14, 14, 15), (10, 12, 11, 11, 14, 13, 14, 15), (10, 11, 12, 11, 14, 14, 13, 15), (16, 17, 17, 17, 18, 18, 18, 19))

NB = 400
NBLK = N // NB
EB = 2000
RBT_W = 656
MSRC_W = 1536

NCHUNK = 14
CH = 768
NPAD = NCHUNK * CH
ACC_ROWS = 832
DUMMY = 831
EPT = E // 16
STRIP = 400
NSTRIP = EPT // STRIP
K = 32
MA_W = 8 * D
MB_W = 4 * D


def _silu(x):
    return x / (1.0 + jnp.exp(-x))


def _init_body(an_ref, es_ref, et_ref, out_ref):
    an = an_ref[...]
    tt = lax.broadcasted_iota(jnp.int32, (1, NUM_TYPES), 1)
    oh = (an == tt).astype(jnp.float32)
    s0 = jnp.dot(oh, es_ref[...], preferred_element_type=jnp.float32)
    s7 = jnp.dot(oh, et_ref[...], preferred_element_type=jnp.float32)
    out_ref[:, 0:D] = s0
    out_ref[:, D:7 * D] = jnp.zeros((NB, 6 * D), jnp.float32)
    out_ref[:, 7 * D:8 * D] = s7


def _init_state(an2, emb_s, emb_t):
    return pl.pallas_call(
        _init_body,
        grid=(NBLK,),
        in_specs=[
            pl.BlockSpec((NB, 1), lambda i: (i, 0)),
            pl.BlockSpec((NUM_TYPES, D), lambda i: (0, 0)),
            pl.BlockSpec((NUM_TYPES, D), lambda i: (0, 0)),
        ],
        out_specs=pl.BlockSpec((NB, 8 * D), lambda i: (i, 0)),
        out_shape=jax.ShapeDtypeStruct((N, 8 * D), jnp.float32),
    )(an2, emb_s, emb_t)


def _rbt_body(el_ref, ev_ref, w_ref, b_ref, out_ref):
    el = el_ref[...]
    r = jnp.maximum(el, 1e-6)
    k = lax.broadcasted_iota(jnp.int32, (1, NUM_RBF), 1).astype(jnp.float32) + 1.0
    freqs = k * (math.pi / R_CUT)
    rbf = jnp.sin(r * freqs) / r
    cutoff = 0.5 * (jnp.cos((math.pi / R_CUT) * el) + 1.0)
    cutoff = cutoff * (el < R_CUT).astype(jnp.float32)
    out640 = (jnp.dot(rbf, w_ref[...], preferred_element_type=jnp.float32)
              + b_ref[...]) * cutoff
    out_ref[:, 0:5 * D] = out640
    out_ref[:, 5 * D:5 * D + 3] = ev_ref[...]
    out_ref[:, 5 * D + 3:RBT_W] = jnp.zeros((EB, RBT_W - 5 * D - 3), jnp.float32)


def _rbt(el2, ev, w, b2):
    return pl.pallas_call(
        _rbt_body,
        grid=(E // EB,),
        in_specs=[
            pl.BlockSpec((EB, 1), lambda i: (i, 0)),
            pl.BlockSpec((EB, 3), lambda i: (i, 0)),
            pl.BlockSpec((NUM_RBF, 5 * D), lambda i: (0, 0)),
            pl.BlockSpec((1, 5 * D), lambda i: (0, 0)),
        ],
        out_specs=pl.BlockSpec((EB, RBT_W), lambda i: (i, 0)),
        out_shape=jax.ShapeDtypeStruct((E, RBT_W), jnp.float32),
    )(el2, ev, w, b2)


def _msrc_body(st_ref, w1_ref, b1_ref, w2_ref, b2_ref, a_ref, b_ref):
    s0 = st_ref[:, 0:D]
    h = _silu(jnp.dot(s0, w1_ref[...], preferred_element_type=jnp.float32)
              + b1_ref[...])
    phi = jnp.dot(h, w2_ref[...], preferred_element_type=jnp.float32) + b2_ref[...]
    a_ref[:, 0:D] = phi[:, 0:D]
    a_ref[:, D:8 * D] = st_ref[:, D:8 * D]
    b_ref[...] = phi[:, D:5 * D]


def _msrc(state, w1, b1, w2, b2):
    return pl.pallas_call(
        _msrc_body,
        grid=(NBLK,),
        in_specs=[
            pl.BlockSpec((NB, 8 * D), lambda i: (i, 0)),
            pl.BlockSpec((D, 5 * D), lambda i: (0, 0)),
            pl.BlockSpec((1, 5 * D), lambda i: (0, 0)),
            pl.BlockSpec((5 * D, 5 * D), lambda i: (0, 0)),
            pl.BlockSpec((1, 5 * D), lambda i: (0, 0)),
        ],
        out_specs=[
            pl.BlockSpec((NB, MA_W), lambda i: (i, 0)),
            pl.BlockSpec((NB, MB_W), lambda i: (i, 0)),
        ],
        out_shape=[
            jax.ShapeDtypeStruct((N, MA_W), jnp.float32),
            jax.ShapeDtypeStruct((N, MB_W), jnp.float32),
        ],
    )(state, w1, b1, w2, b2)


def _edge_body(snd_hbm, rcv_hbm, ma_hbm, mb_hbm, rbt_hbm, agg_hbm,
               acc, sstrip, rstrip, feid, fsnd, frcv,
               ma, mb_, rr, eidx, sidx, ridx,
               semA, semB, semC, semS):
    core = lax.axis_index("c")
    sub = lax.axis_index("s")
    tile_ebase = sub * EPT
    lanes = lax.iota(jnp.int32, 16)

    def issue_gathers(b, nvalid):
        for h in range(K // 16):
            valid = (h * 16 + lanes) < nvalid
            eidx[pl.ds(h * 16, 16)] = jnp.where(
                valid, feid[pl.ds(h * 16, 16)], 0)
            sidx[pl.ds(h * 16, 16)] = jnp.where(
                valid, fsnd[pl.ds(h * 16, 16)], 0)
            ridx[pl.ds(h * 16, 16)] = jnp.where(
                valid, frcv[pl.ds(h * 16, 16)], DUMMY)
        pltpu.async_copy(ma_hbm.at[sidx], ma, semA)
        pltpu.async_copy(mb_hbm.at[sidx], mb_, semB)
        pltpu.async_copy(rbt_hbm.at[eidx], rr, semC)

    def finish_batch(b):
        pltpu.make_async_copy(ma_hbm.at[sidx], ma, semA).wait()
        pltpu.make_async_copy(mb_hbm.at[sidx], mb_, semB).wait()
        pltpu.make_async_copy(rbt_hbm.at[eidx], rr, semC).wait()

        def edge_compute(e, _):
            evv = rr[e, pl.ds(5 * D, 16)]
            ev0 = evv[0]
            ev1 = evv[1]
            ev2 = evv[2]
            for j in range(8):
                o = j * 16
                rs = rr[e, pl.ds(o, 16)]
                rv_ = rr[e, pl.ds(D + o, 16)]
                rd_ = rr[e, pl.ds(2 * D + o, 16)]
                rb_ = rr[e, pl.ds(3 * D + o, 16)]
                rt_ = rr[e, pl.ds(4 * D + o, 16)]
                ps = ma[e, pl.ds(o, 16)]
                sv0 = ma[e, pl.ds(D + o, 16)]
                sv1 = ma[e, pl.ds(2 * D + o, 16)]
                sv2 = ma[e, pl.ds(3 * D + o, 16)]
                sb0 = ma[e, pl.ds(4 * D + o, 16)]
                sb1 = ma[e, pl.ds(5 * D + o, 16)]
                sb2 = ma[e, pl.ds(6 * D + o, 16)]
                st_ = ma[e, pl.ds(7 * D + o, 16)]
                g_v = mb_[e, pl.ds(o, 16)] * rv_
                g_d = mb_[e, pl.ds(D + o, 16)] * rd_
                g_b = mb_[e, pl.ds(2 * D + o, 16)] * rb_
                g_t = mb_[e, pl.ds(3 * D + o, 16)] * rt_
                ma[e, pl.ds(o, 16)] = ps * rs
                ma[e, pl.ds(D + o, 16)] = g_v * sv0 + g_d * ev0
                ma[e, pl.ds(2 * D + o, 16)] = g_v * sv1 + g_d * ev1
                ma[e, pl.ds(3 * D + o, 16)] = g_v * sv2 + g_d * ev2
                ma[e, pl.ds(4 * D + o, 16)] = g_b * sb0
                ma[e, pl.ds(5 * D + o, 16)] = g_b * sb1
                ma[e, pl.ds(6 * D + o, 16)] = g_b * sb2
                ma[e, pl.ds(7 * D + o, 16)] = g_t * st_
            return 0
        lax.fori_loop(0, K, edge_compute, 0)
        pltpu.async_copy(ma, acc.at[ridx], semS, add=True)

    def wait_scatter(b):
        pltpu.make_async_copy(ma, acc.at[ridx], semS).wait()

    def scan_edges(lo, hi):
        def strip_body(s, carry):
            sbase = tile_ebase + s * STRIP
            pltpu.sync_copy(snd_hbm.at[pl.ds(sbase, STRIP)], sstrip)
            pltpu.sync_copy(rcv_hbm.at[pl.ds(sbase, STRIP)], rstrip)

            def vbody(j, carry):
                fill, fc = carry
                rv = rstrip[pl.ds(j * 16, 16)]
                sv = sstrip[pl.ds(j * 16, 16)]
                m = (rv >= lo) & (rv < hi)
                mi = m.astype(jnp.int32)
                pos = fill + plsc.cumsum(mi) - 1
                eidv = sbase + j * 16 + lanes
                plsc.store_scatter(feid, [pos], eidv, mask=m)
                plsc.store_scatter(fsnd, [pos], sv, mask=m)
                plsc.store_scatter(frcv, [pos], rv - lo, mask=m)
                fill = fill + jnp.sum(mi)

                @pl.when(fill >= K)
                def _flush():
                    issue_gathers(0, K)
                    finish_batch(0)
                    wait_scatter(0)
                    feid[pl.ds(0, 16)] = feid[pl.ds(K, 16)]
                    fsnd[pl.ds(0, 16)] = fsnd[pl.ds(K, 16)]
                    frcv[pl.ds(0, 16)] = frcv[pl.ds(K, 16)]
                flushed = fill >= K
                return (jnp.where(flushed, fill - K, fill),
                        jnp.where(flushed, fc + 1, fc))
            return lax.fori_loop(0, STRIP // 16, vbody, carry)
        fill, fc = lax.fori_loop(0, NSTRIP, strip_body,
                                 (jnp.int32(0), jnp.int32(0)))

        @pl.when(fill > 0)
        def _final():
            issue_gathers(0, fill)
            finish_batch(0)
            wait_scatter(0)

    def chunk_body(chunk, _):
        lo = chunk * CH

        @pl.when(core == (chunk % 2))
        def _process():
            def zb(e, _):
                for j in range(64):
                    ma[e, pl.ds(j * 16, 16)] = jnp.zeros((16,), jnp.float32)
                return 0
            lax.fori_loop(0, K, zb, 0)
            zoff = jnp.where(sub < 8, sub * 56, 448 + (sub - 8) * 48)
            pltpu.sync_copy(ma, acc.at[pl.ds(zoff, 32)])
            pltpu.sync_copy(ma.at[pl.ds(0, 16)], acc.at[pl.ds(zoff + 32, 16)])

            @pl.when(sub < 8)
            def _zextra():
                pltpu.sync_copy(ma.at[pl.ds(0, 8)],
                                acc.at[pl.ds(sub * 56 + 48, 8)])
            plsc.subcore_barrier()
            scan_edges(lo, lo + CH)
            plsc.subcore_barrier()

            @pl.when(sub < 8)
            def _writeback():
                pltpu.sync_copy(acc.at[pl.ds(sub * (CH // 8), CH // 8)],
                                agg_hbm.at[pl.ds(lo + sub * (CH // 8), CH // 8)])
            plsc.subcore_barrier()
        return 0

    lax.fori_loop(0, NCHUNK, chunk_body, 0)


def _edge_aggregate(senders, receivers, msrc_a, msrc_b, rbt):
    mesh = plsc.VectorSubcoreMesh(core_axis_name="c", subcore_axis_name="s")
    vm = pltpu.VMEM
    f = pl.kernel(
        _edge_body,
        mesh=mesh,
        compiler_params=pltpu.CompilerParams(use_tc_tiling_on_sc=False,
                                             needs_layout_passes=False),
        out_type=jax.ShapeDtypeStruct((NPAD, 8 * D), jnp.float32),
        scratch_types=(
            [pltpu.VMEM_SHARED((ACC_ROWS, 8 * D), jnp.float32),
             vm((STRIP,), jnp.int32), vm((STRIP,), jnp.int32),
             vm((48,), jnp.int32), vm((48,), jnp.int32), vm((48,), jnp.int32),
             vm((K, MA_W), jnp.float32), vm((K, MB_W), jnp.float32),
             vm((K, RBT_W), jnp.float32), vm((K,), jnp.int32),
             vm((K,), jnp.int32), vm((K,), jnp.int32)]
            + [pltpu.SemaphoreType.DMA] * 4
        ),
    )
    return f(senders, receivers, msrc_a, msrc_b, rbt)


def _node_body(st_ref, agg_ref, an_ref, uw_ref, vw_ref, z1w_ref, z2w_ref,
               g1_ref, g2_ref, w1_ref, b1_ref, w2_ref, b2_ref, out_ref):
    st = [st_ref[:, c * D:(c + 1) * D] + agg_ref[:, c * D:(c + 1) * D]
          for c in range(8)]
    uw = uw_ref[...]
    vw = vw_ref[...]
    U = [jnp.dot(st[c], uw, preferred_element_type=jnp.float32) for c in range(8)]
    V = [jnp.dot(st[c], vw, preferred_element_type=jnp.float32) for c in range(8)]

    def wmp(A, B, w_ref):
        out = [None] * 8
        for i in range(8):
            for j in range(8):
                gp = _GP_IDX[i][j]
                w = w_ref[_W_IDX[i][j]:_W_IDX[i][j] + 1, :]
                term = A[i] * B[j] * w
                c = gp if gp < 8 else gp - 8
                if out[c] is None:
                    out[c] = term if gp < 8 else -term
                else:
                    out[c] = out[c] + term if gp < 8 else out[c] - term
        return out

    Z1 = wmp(U, V, z1w_ref)
    g1 = g1_ref[...]
    Z1l = [jnp.dot(Z1[c], g1, preferred_element_type=jnp.float32) for c in range(8)]
    Z2 = wmp(U, Z1l, z2w_ref)
    g2 = g2_ref[...]
    Z2l = [jnp.dot(Z2[c], g2, preferred_element_type=jnp.float32) for c in range(8)]

    v_norm = jnp.sqrt(V[1] * V[1] + V[2] * V[2] + V[3] * V[3])
    upd_in = jnp.concatenate([st[0], v_norm], axis=1)
    an = an_ref[...]
    a = jnp.zeros((NB, 4 * D), jnp.float32)
    for t in range(NUM_TYPES):
        h1 = _silu(jnp.dot(upd_in, w1_ref[t], preferred_element_type=jnp.float32)
                   + b1_ref[t])
        out_t = jnp.dot(h1, w2_ref[t], preferred_element_type=jnp.float32) + b2_ref[t]
        a = jnp.where(an == t, out_t, a)
    ach = [a[:, q * D:(q + 1) * D] for q in range(4)]
    new = [None] * 8
    new[0] = st[0] + ach[0] * (U[0] + Z1l[0] + Z2l[0])
    for i in range(3):
        new[1 + i] = st[1 + i] + ach[1] * (U[1 + i] + Z1l[1 + i] + Z2l[1 + i])
        new[4 + i] = st[4 + i] + ach[2] * (U[4 + i] + Z1l[4 + i] + Z2l[4 + i])
    new[7] = st[7] + ach[3] * (U[7] + Z1l[7] + Z2l[7])
    out_ref[...] = jnp.concatenate(new, axis=1)


def _node_update(state, agg, an2, uw, vw, z1w, z2w, g1, g2, w1, b1, w2, b2):
    return pl.pallas_call(
        _node_body,
        grid=(NBLK,),
        in_specs=[
            pl.BlockSpec((NB, 8 * D), lambda i: (i, 0)),
            pl.BlockSpec((NB, 8 * D), lambda i: (i, 0)),
            pl.BlockSpec((NB, 1), lambda i: (i, 0)),
            pl.BlockSpec((D, D), lambda i: (0, 0)),
            pl.BlockSpec((D, D), lambda i: (0, 0)),
            pl.BlockSpec((NUM_RBF, D), lambda i: (0, 0)),
            pl.BlockSpec((NUM_RBF, D), lambda i: (0, 0)),
            pl.BlockSpec((D, D), lambda i: (0, 0)),
            pl.BlockSpec((D, D), lambda i: (0, 0)),
            pl.BlockSpec((NUM_TYPES, 2 * D, 4 * D), lambda i: (0, 0, 0)),
            pl.BlockSpec((NUM_TYPES, 1, 4 * D), lambda i: (0, 0, 0)),
            pl.BlockSpec((NUM_TYPES, 4 * D, 4 * D), lambda i: (0, 0, 0)),
            pl.BlockSpec((NUM_TYPES, 1, 4 * D), lambda i: (0, 0, 0)),
        ],
        out_specs=pl.BlockSpec((NB, 8 * D), lambda i: (i, 0)),
        out_shape=jax.ShapeDtypeStruct((N, 8 * D), jnp.float32),
    )(state, agg, an2, uw, vw, z1w, z2w, g1, g2, w1, b1, w2, b2)


def _readout_body(st_ref, co_ref, ngi_ref, out_ref, acc):
    i = pl.program_id(0)

    @pl.when(i == 0)
    def _init():
        acc[...] = jnp.zeros((G, D), jnp.float32)

    st0 = st_ref[:, 0:D]
    q = jnp.sum(st0, axis=1, keepdims=True)
    cols = []
    for ax in range(3):
        m = jnp.sum(st_ref[:, (1 + ax) * D:(2 + ax) * D], axis=1, keepdims=True)
        cols.append(m + q * co_ref[:, ax:ax + 1])
    mu_pad = jnp.concatenate(cols + [jnp.zeros((NB, D - 3), jnp.float32)], axis=1)
    ngi = ngi_ref[0]
    gcol = lax.broadcasted_iota(jnp.int32, (G, 1), 0)
    ohT = (gcol == ngi).astype(jnp.float32)
    acc[...] = acc[...] + jnp.dot(ohT, mu_pad, preferred_element_type=jnp.float32)

    @pl.when(i == NBLK - 1)
    def _fin():
        d3 = acc[:, 0:3]
        out_ref[...] = jnp.sqrt(jnp.sum(d3 * d3, axis=1, keepdims=True))


def _readout(state, coords, ngi3):
    return pl.pallas_call(
        _readout_body,
        grid=(NBLK,),
        in_specs=[
            pl.BlockSpec((NB, 8 * D), lambda i: (i, 0)),
            pl.BlockSpec((NB, 3), lambda i: (i, 0)),
            pl.BlockSpec((1, 1, NB), lambda i: (i, 0, 0)),
        ],
        out_specs=pl.BlockSpec((G, 1), lambda i: (0, 0)),
        out_shape=jax.ShapeDtypeStruct((G, 1), jnp.float32),
        scratch_shapes=[pltpu.VMEM((G, D), jnp.float32)],
    )(state, coords, ngi3)


def kernel(num_nodes, num_graphs, atomic_numbers, edge_list, edge_lengths,
           edge_vectors, node_coordinates, node_graph_index, emb_scalar,
           emb_tri, rbf_W, rbf_b, phi_W1, phi_b1, phi_W2, phi_b2, U_W, V_W,
           Z1w, Z2w, gpZ1_W, gpZ2_W, upd_W1, upd_b1, upd_W2, upd_b2):
    an2 = atomic_numbers.reshape(N, 1)
    ngi3 = node_graph_index.reshape(NBLK, 1, NB)
    senders = edge_list[:, 0]
    receivers = edge_list[:, 1]
    el2 = edge_lengths.reshape(E, 1)

    state = _init_state(an2, emb_scalar, emb_tri)
    rbts = [_rbt(el2, edge_vectors, rbf_W[rr], rbf_b[rr].reshape(1, 5 * D))
            for rr in range(2)]
    for rr in range(2):
        msrc_a, msrc_b = _msrc(state, phi_W1[rr], phi_b1[rr].reshape(1, 5 * D),
                               phi_W2[rr], phi_b2[rr].reshape(1, 5 * D))
        agg = _edge_aggregate(senders, receivers, msrc_a, msrc_b, rbts[rr])
        state = _node_update(
            state, agg, an2, U_W[rr], V_W[rr], Z1w[rr], Z2w[rr],
            gpZ1_W[rr], gpZ2_W[rr], upd_W1[rr],
            upd_b1[rr].reshape(NUM_TYPES, 1, 4 * D), upd_W2[rr],
            upd_b2[rr].reshape(NUM_TYPES, 1, 4 * D))
    return _readout(state, node_coordinates, ngi3)


# speedup vs baseline: 11.8514x; 1.0409x over previous
import functools
import math

import jax
import jax.numpy as jnp
from jax import lax
from jax.experimental import pallas as pl
from jax.experimental.pallas import tpu as pltpu
from jax.experimental.pallas import tpu_sc as plsc

N = 10000
E = 160000
D = 128
G = 64
NUM_RBF = 20
R_CUT = 5.0
NUM_TYPES = 5

_GP_IDX = ((0, 1, 2, 3, 4, 5, 6, 7), (1, 0, 4, 14, 2, 7, 11, 5), (2, 12, 0, 5, 9, 3, 7, 6), (3, 6, 13, 0, 7, 10, 1, 4), (4, 10, 1, 7, 8, 14, 5, 11), (5, 7, 11, 2, 6, 8, 12, 9), (6, 3, 7, 9, 13, 4, 8, 10), (7, 5, 6, 4, 11, 9, 10, 8))
_W_IDX = ((0, 1, 1, 1, 2, 2, 2, 3), (4, 5, 6, 6, 7, 8, 7, 9), (4, 6, 5, 6, 7, 7, 8, 9), (4, 6, 6, 5, 8, 7, 7, 9), (10, 11, 11, 12, 13, 14, 14, 15), (10, 12, 11, 11, 14, 13, 14, 15), (10, 11, 12, 11, 14, 14, 13, 15), (16, 17, 17, 17, 18, 18, 18, 19))

NB = 400
NBLK = N // NB
EB = 2000
RBT_W = 656
MSRC_W = 1536

NCHUNK = 14
CH = 768
NPAD = NCHUNK * CH
ACC_ROWS = 832
DUMMY = 831
EPT = E // 16
STRIP = 2000
NSTRIP = EPT // STRIP
K = 32
MA_W = 8 * D
MB_W = 4 * D


def _silu(x):
    return x / (1.0 + jnp.exp(-x))


def _init_body(an_ref, es_ref, et_ref, out_ref):
    an = an_ref[...]
    tt = lax.broadcasted_iota(jnp.int32, (1, NUM_TYPES), 1)
    oh = (an == tt).astype(jnp.float32)
    s0 = jnp.dot(oh, es_ref[...], preferred_element_type=jnp.float32)
    s7 = jnp.dot(oh, et_ref[...], preferred_element_type=jnp.float32)
    out_ref[:, 0:D] = s0
    out_ref[:, D:7 * D] = jnp.zeros((NB, 6 * D), jnp.float32)
    out_ref[:, 7 * D:8 * D] = s7


def _init_state(an2, emb_s, emb_t):
    return pl.pallas_call(
        _init_body,
        grid=(NBLK,),
        in_specs=[
            pl.BlockSpec((NB, 1), lambda i: (i, 0)),
            pl.BlockSpec((NUM_TYPES, D), lambda i: (0, 0)),
            pl.BlockSpec((NUM_TYPES, D), lambda i: (0, 0)),
        ],
        out_specs=pl.BlockSpec((NB, 8 * D), lambda i: (i, 0)),
        out_shape=jax.ShapeDtypeStruct((N, 8 * D), jnp.float32),
    )(an2, emb_s, emb_t)


def _rbt_body(el_ref, ev_ref, w_ref, b_ref, out_ref):
    el = el_ref[...]
    r = jnp.maximum(el, 1e-6)
    k = lax.broadcasted_iota(jnp.int32, (1, NUM_RBF), 1).astype(jnp.float32) + 1.0
    freqs = k * (math.pi / R_CUT)
    rbf = jnp.sin(r * freqs) / r
    cutoff = 0.5 * (jnp.cos((math.pi / R_CUT) * el) + 1.0)
    cutoff = cutoff * (el < R_CUT).astype(jnp.float32)
    out640 = (jnp.dot(rbf, w_ref[...], preferred_element_type=jnp.float32)
              + b_ref[...]) * cutoff
    out_ref[:, 0:5 * D] = out640
    out_ref[:, 5 * D:5 * D + 3] = ev_ref[...]
    out_ref[:, 5 * D + 3:RBT_W] = jnp.zeros((EB, RBT_W - 5 * D - 3), jnp.float32)


def _rbt(el2, ev, w, b2):
    return pl.pallas_call(
        _rbt_body,
        grid=(E // EB,),
        in_specs=[
            pl.BlockSpec((EB, 1), lambda i: (i, 0)),
            pl.BlockSpec((EB, 3), lambda i: (i, 0)),
            pl.BlockSpec((NUM_RBF, 5 * D), lambda i: (0, 0)),
            pl.BlockSpec((1, 5 * D), lambda i: (0, 0)),
        ],
        out_specs=pl.BlockSpec((EB, RBT_W), lambda i: (i, 0)),
        out_shape=jax.ShapeDtypeStruct((E, RBT_W), jnp.float32),
    )(el2, ev, w, b2)


def _msrc_body(st_ref, w1_ref, b1_ref, w2_ref, b2_ref, a_ref, b_ref):
    s0 = st_ref[:, 0:D]
    h = _silu(jnp.dot(s0, w1_ref[...], preferred_element_type=jnp.float32)
              + b1_ref[...])
    phi = jnp.dot(h, w2_ref[...], preferred_element_type=jnp.float32) + b2_ref[...]
    a_ref[:, 0:D] = phi[:, 0:D]
    a_ref[:, D:8 * D] = st_ref[:, D:8 * D]
    b_ref[...] = phi[:, D:5 * D]


def _msrc(state, w1, b1, w2, b2):
    return pl.pallas_call(
        _msrc_body,
        grid=(NBLK,),
        in_specs=[
            pl.BlockSpec((NB, 8 * D), lambda i: (i, 0)),
            pl.BlockSpec((D, 5 * D), lambda i: (0, 0)),
            pl.BlockSpec((1, 5 * D), lambda i: (0, 0)),
            pl.BlockSpec((5 * D, 5 * D), lambda i: (0, 0)),
            pl.BlockSpec((1, 5 * D), lambda i: (0, 0)),
        ],
        out_specs=[
            pl.BlockSpec((NB, MA_W), lambda i: (i, 0)),
            pl.BlockSpec((NB, MB_W), lambda i: (i, 0)),
        ],
        out_shape=[
            jax.ShapeDtypeStruct((N, MA_W), jnp.float32),
            jax.ShapeDtypeStruct((N, MB_W), jnp.float32),
        ],
    )(state, w1, b1, w2, b2)


def _edge_body(snd_hbm, rcv_hbm, ma_hbm, mb_hbm, rbt_hbm, agg_hbm,
               acc, sstrip, rstrip, feid, fsnd, frcv,
               ma, mb_, rr, eidx, sidx, ridx,
               semA, semB, semC, semS):
    core = lax.axis_index("c")
    sub = lax.axis_index("s")
    tile_ebase = sub * EPT
    lanes = lax.iota(jnp.int32, 16)

    def issue_gathers(b, nvalid):
        for h in range(K // 16):
            valid = (h * 16 + lanes) < nvalid
            eidx[pl.ds(h * 16, 16)] = jnp.where(
                valid, feid[pl.ds(h * 16, 16)], 0)
            sidx[pl.ds(h * 16, 16)] = jnp.where(
                valid, fsnd[pl.ds(h * 16, 16)], 0)
            ridx[pl.ds(h * 16, 16)] = jnp.where(
                valid, frcv[pl.ds(h * 16, 16)], DUMMY)
        pltpu.async_copy(ma_hbm.at[sidx], ma, semA)
        pltpu.async_copy(mb_hbm.at[sidx], mb_, semB)
        pltpu.async_copy(rbt_hbm.at[eidx], rr, semC)

    def finish_batch(b):
        pltpu.make_async_copy(ma_hbm.at[sidx], ma, semA).wait()
        pltpu.make_async_copy(mb_hbm.at[sidx], mb_, semB).wait()
        pltpu.make_async_copy(rbt_hbm.at[eidx], rr, semC).wait()

        def edge_compute(e, _):
            evv = rr[e, pl.ds(5 * D, 16)]
            ev0 = evv[0]
            ev1 = evv[1]
            ev2 = evv[2]
            for j in range(8):
                o = j * 16
                rs = rr[e, pl.ds(o, 16)]
                rv_ = rr[e, pl.ds(D + o, 16)]
                rd_ = rr[e, pl.ds(2 * D + o, 16)]
                rb_ = rr[e, pl.ds(3 * D + o, 16)]
                rt_ = rr[e, pl.ds(4 * D + o, 16)]
                ps = ma[e, pl.ds(o, 16)]
                sv0 = ma[e, pl.ds(D + o, 16)]
                sv1 = ma[e, pl.ds(2 * D + o, 16)]
                sv2 = ma[e, pl.ds(3 * D + o, 16)]
                sb0 = ma[e, pl.ds(4 * D + o, 16)]
                sb1 = ma[e, pl.ds(5 * D + o, 16)]
                sb2 = ma[e, pl.ds(6 * D + o, 16)]
                st_ = ma[e, pl.ds(7 * D + o, 16)]
                g_v = mb_[e, pl.ds(o, 16)] * rv_
                g_d = mb_[e, pl.ds(D + o, 16)] * rd_
                g_b = mb_[e, pl.ds(2 * D + o, 16)] * rb_
                g_t = mb_[e, pl.ds(3 * D + o, 16)] * rt_
                ma[e, pl.ds(o, 16)] = ps * rs
                ma[e, pl.ds(D + o, 16)] = g_v * sv0 + g_d * ev0
                ma[e, pl.ds(2 * D + o, 16)] = g_v * sv1 + g_d * ev1
                ma[e, pl.ds(3 * D + o, 16)] = g_v * sv2 + g_d * ev2
                ma[e, pl.ds(4 * D + o, 16)] = g_b * sb0
                ma[e, pl.ds(5 * D + o, 16)] = g_b * sb1
                ma[e, pl.ds(6 * D + o, 16)] = g_b * sb2
                ma[e, pl.ds(7 * D + o, 16)] = g_t * st_
            return 0
        lax.fori_loop(0, K, edge_compute, 0)
        pltpu.async_copy(ma, acc.at[ridx], semS, add=True)

    def wait_scatter(b):
        pltpu.make_async_copy(ma, acc.at[ridx], semS).wait()

    def scan_edges(lo, hi):
        def strip_body(s, carry):
            sbase = tile_ebase + s * STRIP
            pltpu.sync_copy(snd_hbm.at[pl.ds(sbase, STRIP)], sstrip)
            pltpu.sync_copy(rcv_hbm.at[pl.ds(sbase, STRIP)], rstrip)

            def vbody(j, carry):
                fill, fc = carry
                rv = rstrip[pl.ds(j * 16, 16)]
                sv = sstrip[pl.ds(j * 16, 16)]
                m = (rv >= lo) & (rv < hi)
                mi = m.astype(jnp.int32)
                pos = fill + plsc.cumsum(mi) - 1
                eidv = sbase + j * 16 + lanes
                plsc.store_scatter(feid, [pos], eidv, mask=m)
                plsc.store_scatter(fsnd, [pos], sv, mask=m)
                plsc.store_scatter(frcv, [pos], rv - lo, mask=m)
                fill = fill + jnp.sum(mi)

                @pl.when(fill >= K)
                def _flush():
                    issue_gathers(0, K)
                    finish_batch(0)
                    wait_scatter(0)
                    feid[pl.ds(0, 16)] = feid[pl.ds(K, 16)]
                    fsnd[pl.ds(0, 16)] = fsnd[pl.ds(K, 16)]
                    frcv[pl.ds(0, 16)] = frcv[pl.ds(K, 16)]
                flushed = fill >= K
                return (jnp.where(flushed, fill - K, fill),
                        jnp.where(flushed, fc + 1, fc))
            return lax.fori_loop(0, STRIP // 16, vbody, carry)
        fill, fc = lax.fori_loop(0, NSTRIP, strip_body,
                                 (jnp.int32(0), jnp.int32(0)))

        @pl.when(fill > 0)
        def _final():
            issue_gathers(0, fill)
            finish_batch(0)
            wait_scatter(0)

    def chunk_body(chunk, _):
        lo = chunk * CH

        @pl.when(core == (chunk % 2))
        def _process():
            def zb(e, _):
                for j in range(64):
                    ma[e, pl.ds(j * 16, 16)] = jnp.zeros((16,), jnp.float32)
                return 0
            lax.fori_loop(0, K, zb, 0)
            zoff = jnp.where(sub < 8, sub * 56, 448 + (sub - 8) * 48)
            pltpu.sync_copy(ma, acc.at[pl.ds(zoff, 32)])
            pltpu.sync_copy(ma.at[pl.ds(0, 16)], acc.at[pl.ds(zoff + 32, 16)])

            @pl.when(sub < 8)
            def _zextra():
                pltpu.sync_copy(ma.at[pl.ds(0, 8)],
                                acc.at[pl.ds(sub * 56 + 48, 8)])
            plsc.subcore_barrier()
            scan_edges(lo, lo + CH)
            plsc.subcore_barrier()

            @pl.when(sub < 8)
            def _writeback():
                pltpu.sync_copy(acc.at[pl.ds(sub * (CH // 8), CH // 8)],
                                agg_hbm.at[pl.ds(lo + sub * (CH // 8), CH // 8)])
            plsc.subcore_barrier()
        return 0

    lax.fori_loop(0, NCHUNK, chunk_body, 0)


def _edge_aggregate(senders, receivers, msrc_a, msrc_b, rbt):
    mesh = plsc.VectorSubcoreMesh(core_axis_name="c", subcore_axis_name="s")
    vm = pltpu.VMEM
    f = pl.kernel(
        _edge_body,
        mesh=mesh,
        compiler_params=pltpu.CompilerParams(use_tc_tiling_on_sc=False,
                                             needs_layout_passes=False),
        out_type=jax.ShapeDtypeStruct((NPAD, 8 * D), jnp.float32),
        scratch_types=(
            [pltpu.VMEM_SHARED((ACC_ROWS, 8 * D), jnp.float32),
             vm((STRIP,), jnp.int32), vm((STRIP,), jnp.int32),
             vm((48,), jnp.int32), vm((48,), jnp.int32), vm((48,), jnp.int32),
             vm((K, MA_W), jnp.float32), vm((K, MB_W), jnp.float32),
             vm((K, RBT_W), jnp.float32), vm((K,), jnp.int32),
             vm((K,), jnp.int32), vm((K,), jnp.int32)]
            + [pltpu.SemaphoreType.DMA] * 4
        ),
    )
    return f(senders, receivers, msrc_a, msrc_b, rbt)


def _node_body(st_ref, agg_ref, an_ref, uw_ref, vw_ref, z1w_ref, z2w_ref,
               g1_ref, g2_ref, w1_ref, b1_ref, w2_ref, b2_ref, out_ref):
    st = [st_ref[:, c * D:(c + 1) * D] + agg_ref[:, c * D:(c + 1) * D]
          for c in range(8)]
    uw = uw_ref[...]
    vw = vw_ref[...]
    U = [jnp.dot(st[c], uw, preferred_element_type=jnp.float32) for c in range(8)]
    V = [jnp.dot(st[c], vw, preferred_element_type=jnp.float32) for c in range(8)]

    def wmp(A, B, w_ref):
        out = [None] * 8
        for i in range(8):
            for j in range(8):
                gp = _GP_IDX[i][j]
                w = w_ref[_W_IDX[i][j]:_W_IDX[i][j] + 1, :]
                term = A[i] * B[j] * w
                c = gp if gp < 8 else gp - 8
                if out[c] is None:
                    out[c] = term if gp < 8 else -term
                else:
                    out[c] = out[c] + term if gp < 8 else out[c] - term
        return out

    Z1 = wmp(U, V, z1w_ref)
    g1 = g1_ref[...]
    Z1l = [jnp.dot(Z1[c], g1, preferred_element_type=jnp.float32) for c in range(8)]
    Z2 = wmp(U, Z1l, z2w_ref)
    g2 = g2_ref[...]
    Z2l = [jnp.dot(Z2[c], g2, preferred_element_type=jnp.float32) for c in range(8)]

    v_norm = jnp.sqrt(V[1] * V[1] + V[2] * V[2] + V[3] * V[3])
    upd_in = jnp.concatenate([st[0], v_norm], axis=1)
    an = an_ref[...]
    a = jnp.zeros((NB, 4 * D), jnp.float32)
    for t in range(NUM_TYPES):
        h1 = _silu(jnp.dot(upd_in, w1_ref[t], preferred_element_type=jnp.float32)
                   + b1_ref[t])
        out_t = jnp.dot(h1, w2_ref[t], preferred_element_type=jnp.float32) + b2_ref[t]
        a = jnp.where(an == t, out_t, a)
    ach = [a[:, q * D:(q + 1) * D] for q in range(4)]
    new = [None] * 8
    new[0] = st[0] + ach[0] * (U[0] + Z1l[0] + Z2l[0])
    for i in range(3):
        new[1 + i] = st[1 + i] + ach[1] * (U[1 + i] + Z1l[1 + i] + Z2l[1 + i])
        new[4 + i] = st[4 + i] + ach[2] * (U[4 + i] + Z1l[4 + i] + Z2l[4 + i])
    new[7] = st[7] + ach[3] * (U[7] + Z1l[7] + Z2l[7])
    out_ref[...] = jnp.concatenate(new, axis=1)


def _node_update(state, agg, an2, uw, vw, z1w, z2w, g1, g2, w1, b1, w2, b2):
    return pl.pallas_call(
        _node_body,
        grid=(NBLK,),
        in_specs=[
            pl.BlockSpec((NB, 8 * D), lambda i: (i, 0)),
            pl.BlockSpec((NB, 8 * D), lambda i: (i, 0)),
            pl.BlockSpec((NB, 1), lambda i: (i, 0)),
            pl.BlockSpec((D, D), lambda i: (0, 0)),
            pl.BlockSpec((D, D), lambda i: (0, 0)),
            pl.BlockSpec((NUM_RBF, D), lambda i: (0, 0)),
            pl.BlockSpec((NUM_RBF, D), lambda i: (0, 0)),
            pl.BlockSpec((D, D), lambda i: (0, 0)),
            pl.BlockSpec((D, D), lambda i: (0, 0)),
            pl.BlockSpec((NUM_TYPES, 2 * D, 4 * D), lambda i: (0, 0, 0)),
            pl.BlockSpec((NUM_TYPES, 1, 4 * D), lambda i: (0, 0, 0)),
            pl.BlockSpec((NUM_TYPES, 4 * D, 4 * D), lambda i: (0, 0, 0)),
            pl.BlockSpec((NUM_TYPES, 1, 4 * D), lambda i: (0, 0, 0)),
        ],
        out_specs=pl.BlockSpec((NB, 8 * D), lambda i: (i, 0)),
        out_shape=jax.ShapeDtypeStruct((N, 8 * D), jnp.float32),
    )(state, agg, an2, uw, vw, z1w, z2w, g1, g2, w1, b1, w2, b2)


def _readout_body(st_ref, co_ref, ngi_ref, out_ref, acc):
    i = pl.program_id(0)

    @pl.when(i == 0)
    def _init():
        acc[...] = jnp.zeros((G, D), jnp.float32)

    st0 = st_ref[:, 0:D]
    q = jnp.sum(st0, axis=1, keepdims=True)
    cols = []
    for ax in range(3):
        m = jnp.sum(st_ref[:, (1 + ax) * D:(2 + ax) * D], axis=1, keepdims=True)
        cols.append(m + q * co_ref[:, ax:ax + 1])
    mu_pad = jnp.concatenate(cols + [jnp.zeros((NB, D - 3), jnp.float32)], axis=1)
    ngi = ngi_ref[0]
    gcol = lax.broadcasted_iota(jnp.int32, (G, 1), 0)
    ohT = (gcol == ngi).astype(jnp.float32)
    acc[...] = acc[...] + jnp.dot(ohT, mu_pad, preferred_element_type=jnp.float32)

    @pl.when(i == NBLK - 1)
    def _fin():
        d3 = acc[:, 0:3]
        out_ref[...] = jnp.sqrt(jnp.sum(d3 * d3, axis=1, keepdims=True))


def _readout(state, coords, ngi3):
    return pl.pallas_call(
        _readout_body,
        grid=(NBLK,),
        in_specs=[
            pl.BlockSpec((NB, 8 * D), lambda i: (i, 0)),
            pl.BlockSpec((NB, 3), lambda i: (i, 0)),
            pl.BlockSpec((1, 1, NB), lambda i: (i, 0, 0)),
        ],
        out_specs=pl.BlockSpec((G, 1), lambda i: (0, 0)),
        out_shape=jax.ShapeDtypeStruct((G, 1), jnp.float32),
        scratch_shapes=[pltpu.VMEM((G, D), jnp.float32)],
    )(state, coords, ngi3)


def kernel(num_nodes, num_graphs, atomic_numbers, edge_list, edge_lengths,
           edge_vectors, node_coordinates, node_graph_index, emb_scalar,
           emb_tri, rbf_W, rbf_b, phi_W1, phi_b1, phi_W2, phi_b2, U_W, V_W,
           Z1w, Z2w, gpZ1_W, gpZ2_W, upd_W1, upd_b1, upd_W2, upd_b2):
    an2 = atomic_numbers.reshape(N, 1)
    ngi3 = node_graph_index.reshape(NBLK, 1, NB)
    senders = edge_list[:, 0]
    receivers = edge_list[:, 1]
    el2 = edge_lengths.reshape(E, 1)

    state = _init_state(an2, emb_scalar, emb_tri)
    rbts = [_rbt(el2, edge_vectors, rbf_W[rr], rbf_b[rr].reshape(1, 5 * D))
            for rr in range(2)]
    for rr in range(2):
        msrc_a, msrc_b = _msrc(state, phi_W1[rr], phi_b1[rr].reshape(1, 5 * D),
                               phi_W2[rr], phi_b2[rr].reshape(1, 5 * D))
        agg = _edge_aggregate(senders, receivers, msrc_a, msrc_b, rbts[rr])
        state = _node_update(
            state, agg, an2, U_W[rr], V_W[rr], Z1w[rr], Z2w[rr],
            gpZ1_W[rr], gpZ2_W[rr], upd_W1[rr],
            upd_b1[rr].reshape(NUM_TYPES, 1, 4 * D), upd_W2[rr],
            upd_b2[rr].reshape(NUM_TYPES, 1, 4 * D))
    return _readout(state, node_coordinates, ngi3)


# speedup vs baseline: 11.9430x; 1.0077x over previous
import functools
import math

import jax
import jax.numpy as jnp
from jax import lax
from jax.experimental import pallas as pl
from jax.experimental.pallas import tpu as pltpu
from jax.experimental.pallas import tpu_sc as plsc

N = 10000
E = 160000
D = 128
G = 64
NUM_RBF = 20
R_CUT = 5.0
NUM_TYPES = 5

_GP_IDX = ((0, 1, 2, 3, 4, 5, 6, 7), (1, 0, 4, 14, 2, 7, 11, 5), (2, 12, 0, 5, 9, 3, 7, 6), (3, 6, 13, 0, 7, 10, 1, 4), (4, 10, 1, 7, 8, 14, 5, 11), (5, 7, 11, 2, 6, 8, 12, 9), (6, 3, 7, 9, 13, 4, 8, 10), (7, 5, 6, 4, 11, 9, 10, 8))
_W_IDX = ((0, 1, 1, 1, 2, 2, 2, 3), (4, 5, 6, 6, 7, 8, 7, 9), (4, 6, 5, 6, 7, 7, 8, 9), (4, 6, 6, 5, 8, 7, 7, 9), (10, 11, 11, 12, 13, 14, 14, 15), (10, 12, 11, 11, 14, 13, 14, 15), (10, 11, 12, 11, 14, 14, 13, 15), (16, 17, 17, 17, 18, 18, 18, 19))

NB = 400
NBLK = N // NB
EB = 2000
RBT_W = 656
MSRC_W = 1536

NCHUNK = 14
CH = 768
NPAD = NCHUNK * CH
ACC_ROWS = 832
DUMMY = 831
EPT = E // 16
STRIP = 2000
NSTRIP = EPT // STRIP
K = 32
MA_W = 8 * D
MB_W = 4 * D


def _silu(x):
    return x / (1.0 + jnp.exp(-x))


def _phi_tables(s0, st_rest, w1_ref, b1_ref, w2_ref, b2_ref, a_ref, b_ref):
    h = _silu(jnp.dot(s0, w1_ref[...], preferred_element_type=jnp.float32)
              + b1_ref[...])
    phi = jnp.dot(h, w2_ref[...], preferred_element_type=jnp.float32) + b2_ref[...]
    a_ref[:, 0:D] = phi[:, 0:D]
    a_ref[:, D:8 * D] = st_rest
    b_ref[...] = phi[:, D:5 * D]


def _init_body(an_ref, es_ref, et_ref, w1_ref, b1_ref, w2_ref, b2_ref,
               out_ref, a_ref, b_ref):
    an = an_ref[...]
    tt = lax.broadcasted_iota(jnp.int32, (1, NUM_TYPES), 1)
    oh = (an == tt).astype(jnp.float32)
    s0 = jnp.dot(oh, es_ref[...], preferred_element_type=jnp.float32)
    s7 = jnp.dot(oh, et_ref[...], preferred_element_type=jnp.float32)
    zeros6 = jnp.zeros((NB, 6 * D), jnp.float32)
    out_ref[:, 0:D] = s0
    out_ref[:, D:7 * D] = zeros6
    out_ref[:, 7 * D:8 * D] = s7
    st_rest = jnp.concatenate([zeros6, s7], axis=1)
    _phi_tables(s0, st_rest, w1_ref, b1_ref, w2_ref, b2_ref, a_ref, b_ref)


def _init_state(an2, emb_s, emb_t, w1, b1, w2, b2):
    return pl.pallas_call(
        _init_body,
        grid=(NBLK,),
        in_specs=[
            pl.BlockSpec((NB, 1), lambda i: (i, 0)),
            pl.BlockSpec((NUM_TYPES, D), lambda i: (0, 0)),
            pl.BlockSpec((NUM_TYPES, D), lambda i: (0, 0)),
            pl.BlockSpec((D, 5 * D), lambda i: (0, 0)),
            pl.BlockSpec((1, 5 * D), lambda i: (0, 0)),
            pl.BlockSpec((5 * D, 5 * D), lambda i: (0, 0)),
            pl.BlockSpec((1, 5 * D), lambda i: (0, 0)),
        ],
        out_specs=[
            pl.BlockSpec((NB, 8 * D), lambda i: (i, 0)),
            pl.BlockSpec((NB, MA_W), lambda i: (i, 0)),
            pl.BlockSpec((NB, MB_W), lambda i: (i, 0)),
        ],
        out_shape=[
            jax.ShapeDtypeStruct((N, 8 * D), jnp.float32),
            jax.ShapeDtypeStruct((N, MA_W), jnp.float32),
            jax.ShapeDtypeStruct((N, MB_W), jnp.float32),
        ],
    )(an2, emb_s, emb_t, w1, b1, w2, b2)


def _rbt_body(el_ref, ev_ref, w_ref, b_ref, out_ref):
    el = el_ref[...]
    r = jnp.maximum(el, 1e-6)
    k = lax.broadcasted_iota(jnp.int32, (1, NUM_RBF), 1).astype(jnp.float32) + 1.0
    freqs = k * (math.pi / R_CUT)
    rbf = jnp.sin(r * freqs) / r
    cutoff = 0.5 * (jnp.cos((math.pi / R_CUT) * el) + 1.0)
    cutoff = cutoff * (el < R_CUT).astype(jnp.float32)
    out640 = (jnp.dot(rbf, w_ref[...], preferred_element_type=jnp.float32)
              + b_ref[...]) * cutoff
    out_ref[:, 0:5 * D] = out640
    out_ref[:, 5 * D:5 * D + 3] = ev_ref[...]
    out_ref[:, 5 * D + 3:RBT_W] = jnp.zeros((EB, RBT_W - 5 * D - 3), jnp.float32)


def _rbt(el2, ev, w, b2):
    return pl.pallas_call(
        _rbt_body,
        grid=(E // EB,),
        in_specs=[
            pl.BlockSpec((EB, 1), lambda i: (i, 0)),
            pl.BlockSpec((EB, 3), lambda i: (i, 0)),
            pl.BlockSpec((NUM_RBF, 5 * D), lambda i: (0, 0)),
            pl.BlockSpec((1, 5 * D), lambda i: (0, 0)),
        ],
        out_specs=pl.BlockSpec((EB, RBT_W), lambda i: (i, 0)),
        out_shape=jax.ShapeDtypeStruct((E, RBT_W), jnp.float32),
    )(el2, ev, w, b2)


def _edge_body(snd_hbm, rcv_hbm, ma_hbm, mb_hbm, rbt_hbm, agg_hbm,
               acc, sstrip, rstrip, feid, fsnd, frcv,
               ma, mb_, rr, eidx, sidx, ridx,
               semA, semB, semC, semS):
    core = lax.axis_index("c")
    sub = lax.axis_index("s")
    tile_ebase = sub * EPT
    lanes = lax.iota(jnp.int32, 16)

    def issue_gathers(b, nvalid):
        for h in range(K // 16):
            valid = (h * 16 + lanes) < nvalid
            eidx[pl.ds(h * 16, 16)] = jnp.where(
                valid, feid[pl.ds(h * 16, 16)], 0)
            sidx[pl.ds(h * 16, 16)] = jnp.where(
                valid, fsnd[pl.ds(h * 16, 16)], 0)
            ridx[pl.ds(h * 16, 16)] = jnp.where(
                valid, frcv[pl.ds(h * 16, 16)], DUMMY)
        pltpu.async_copy(ma_hbm.at[sidx], ma, semA)
        pltpu.async_copy(mb_hbm.at[sidx], mb_, semB)
        pltpu.async_copy(rbt_hbm.at[eidx], rr, semC)

    def finish_batch(b):
        pltpu.make_async_copy(ma_hbm.at[sidx], ma, semA).wait()
        pltpu.make_async_copy(mb_hbm.at[sidx], mb_, semB).wait()
        pltpu.make_async_copy(rbt_hbm.at[eidx], rr, semC).wait()

        def edge_compute(e, _):
            evv = rr[e, pl.ds(5 * D, 16)]
            ev0 = evv[0]
            ev1 = evv[1]
            ev2 = evv[2]
            for j in range(8):
                o = j * 16
                rs = rr[e, pl.ds(o, 16)]
                rv_ = rr[e, pl.ds(D + o, 16)]
                rd_ = rr[e, pl.ds(2 * D + o, 16)]
                rb_ = rr[e, pl.ds(3 * D + o, 16)]
                rt_ = rr[e, pl.ds(4 * D + o, 16)]
                ps = ma[e, pl.ds(o, 16)]
                sv0 = ma[e, pl.ds(D + o, 16)]
                sv1 = ma[e, pl.ds(2 * D + o, 16)]
                sv2 = ma[e, pl.ds(3 * D + o, 16)]
                sb0 = ma[e, pl.ds(4 * D + o, 16)]
                sb1 = ma[e, pl.ds(5 * D + o, 16)]
                sb2 = ma[e, pl.ds(6 * D + o, 16)]
                st_ = ma[e, pl.ds(7 * D + o, 16)]
                g_v = mb_[e, pl.ds(o, 16)] * rv_
                g_d = mb_[e, pl.ds(D + o, 16)] * rd_
                g_b = mb_[e, pl.ds(2 * D + o, 16)] * rb_
                g_t = mb_[e, pl.ds(3 * D + o, 16)] * rt_
                ma[e, pl.ds(o, 16)] = ps * rs
                ma[e, pl.ds(D + o, 16)] = g_v * sv0 + g_d * ev0
                ma[e, pl.ds(2 * D + o, 16)] = g_v * sv1 + g_d * ev1
                ma[e, pl.ds(3 * D + o, 16)] = g_v * sv2 + g_d * ev2
                ma[e, pl.ds(4 * D + o, 16)] = g_b * sb0
                ma[e, pl.ds(5 * D + o, 16)] = g_b * sb1
                ma[e, pl.ds(6 * D + o, 16)] = g_b * sb2
                ma[e, pl.ds(7 * D + o, 16)] = g_t * st_
            return 0
        lax.fori_loop(0, K, edge_compute, 0)
        pltpu.async_copy(ma, acc.at[ridx], semS, add=True)

    def wait_scatter(b):
        pltpu.make_async_copy(ma, acc.at[ridx], semS).wait()

    def scan_edges(lo, hi):
        def strip_body(s, carry):
            sbase = tile_ebase + s * STRIP
            pltpu.sync_copy(snd_hbm.at[pl.ds(sbase, STRIP)], sstrip)
            pltpu.sync_copy(rcv_hbm.at[pl.ds(sbase, STRIP)], rstrip)

            def vbody(j, carry):
                fill, fc = carry
                rv = rstrip[pl.ds(j * 16, 16)]
                sv = sstrip[pl.ds(j * 16, 16)]
                m = (rv >= lo) & (rv < hi)
                mi = m.astype(jnp.int32)
                pos = fill + plsc.cumsum(mi) - 1
                eidv = sbase + j * 16 + lanes
                plsc.store_scatter(feid, [pos], eidv, mask=m)
                plsc.store_scatter(fsnd, [pos], sv, mask=m)
                plsc.store_scatter(frcv, [pos], rv - lo, mask=m)
                fill = fill + jnp.sum(mi)

                @pl.when(fill >= K)
                def _flush():
                    issue_gathers(0, K)
                    finish_batch(0)
                    wait_scatter(0)
                    feid[pl.ds(0, 16)] = feid[pl.ds(K, 16)]
                    fsnd[pl.ds(0, 16)] = fsnd[pl.ds(K, 16)]
                    frcv[pl.ds(0, 16)] = frcv[pl.ds(K, 16)]
                flushed = fill >= K
                return (jnp.where(flushed, fill - K, fill),
                        jnp.where(flushed, fc + 1, fc))
            return lax.fori_loop(0, STRIP // 16, vbody, carry)
        fill, fc = lax.fori_loop(0, NSTRIP, strip_body,
                                 (jnp.int32(0), jnp.int32(0)))

        @pl.when(fill > 0)
        def _final():
            issue_gathers(0, fill)
            finish_batch(0)
            wait_scatter(0)

    def chunk_body(chunk, _):
        lo = chunk * CH

        @pl.when(core == (chunk % 2))
        def _process():
            def zb(e, _):
                for j in range(64):
                    ma[e, pl.ds(j * 16, 16)] = jnp.zeros((16,), jnp.float32)
                return 0
            lax.fori_loop(0, K, zb, 0)
            zoff = jnp.where(sub < 8, sub * 56, 448 + (sub - 8) * 48)
            pltpu.sync_copy(ma, acc.at[pl.ds(zoff, 32)])
            pltpu.sync_copy(ma.at[pl.ds(0, 16)], acc.at[pl.ds(zoff + 32, 16)])

            @pl.when(sub < 8)
            def _zextra():
                pltpu.sync_copy(ma.at[pl.ds(0, 8)],
                                acc.at[pl.ds(sub * 56 + 48, 8)])
            plsc.subcore_barrier()
            scan_edges(lo, lo + CH)
            plsc.subcore_barrier()

            @pl.when(sub < 8)
            def _writeback():
                pltpu.sync_copy(acc.at[pl.ds(sub * (CH // 8), CH // 8)],
                                agg_hbm.at[pl.ds(lo + sub * (CH // 8), CH // 8)])
            plsc.subcore_barrier()
        return 0

    lax.fori_loop(0, NCHUNK, chunk_body, 0)


def _edge_aggregate(senders, receivers, msrc_a, msrc_b, rbt):
    mesh = plsc.VectorSubcoreMesh(core_axis_name="c", subcore_axis_name="s")
    vm = pltpu.VMEM
    f = pl.kernel(
        _edge_body,
        mesh=mesh,
        compiler_params=pltpu.CompilerParams(use_tc_tiling_on_sc=False,
                                             needs_layout_passes=False),
        out_type=jax.ShapeDtypeStruct((NPAD, 8 * D), jnp.float32),
        scratch_types=(
            [pltpu.VMEM_SHARED((ACC_ROWS, 8 * D), jnp.float32),
             vm((STRIP,), jnp.int32), vm((STRIP,), jnp.int32),
             vm((48,), jnp.int32), vm((48,), jnp.int32), vm((48,), jnp.int32),
             vm((K, MA_W), jnp.float32), vm((K, MB_W), jnp.float32),
             vm((K, RBT_W), jnp.float32), vm((K,), jnp.int32),
             vm((K,), jnp.int32), vm((K,), jnp.int32)]
            + [pltpu.SemaphoreType.DMA] * 4
        ),
    )
    return f(senders, receivers, msrc_a, msrc_b, rbt)


def _node_core(st_ref, agg_ref, an_ref, uw_ref, vw_ref, z1w_ref, z2w_ref,
               g1_ref, g2_ref, w1_ref, b1_ref, w2_ref, b2_ref):
    st = [st_ref[:, c * D:(c + 1) * D] + agg_ref[:, c * D:(c + 1) * D]
          for c in range(8)]
    uw = uw_ref[...]
    vw = vw_ref[...]
    U = [jnp.dot(st[c], uw, preferred_element_type=jnp.float32) for c in range(8)]
    V = [jnp.dot(st[c], vw, preferred_element_type=jnp.float32) for c in range(8)]

    def wmp(A, B, w_ref):
        out = [None] * 8
        for i in range(8):
            for j in range(8):
                gp = _GP_IDX[i][j]
                w = w_ref[_W_IDX[i][j]:_W_IDX[i][j] + 1, :]
                term = A[i] * B[j] * w
                c = gp if gp < 8 else gp - 8
                if out[c] is None:
                    out[c] = term if gp < 8 else -term
                else:
                    out[c] = out[c] + term if gp < 8 else out[c] - term
        return out

    Z1 = wmp(U, V, z1w_ref)
    g1 = g1_ref[...]
    Z1l = [jnp.dot(Z1[c], g1, preferred_element_type=jnp.float32) for c in range(8)]
    Z2 = wmp(U, Z1l, z2w_ref)
    g2 = g2_ref[...]
    Z2l = [jnp.dot(Z2[c], g2, preferred_element_type=jnp.float32) for c in range(8)]

    v_norm = jnp.sqrt(V[1] * V[1] + V[2] * V[2] + V[3] * V[3])
    upd_in = jnp.concatenate([st[0], v_norm], axis=1)
    an = an_ref[...]
    a = jnp.zeros((NB, 4 * D), jnp.float32)
    for t in range(NUM_TYPES):
        h1 = _silu(jnp.dot(upd_in, w1_ref[t], preferred_element_type=jnp.float32)
                   + b1_ref[t])
        out_t = jnp.dot(h1, w2_ref[t], preferred_element_type=jnp.float32) + b2_ref[t]
        a = jnp.where(an == t, out_t, a)
    ach = [a[:, q * D:(q + 1) * D] for q in range(4)]
    new = [None] * 8
    new[0] = st[0] + ach[0] * (U[0] + Z1l[0] + Z2l[0])
    for i in range(3):
        new[1 + i] = st[1 + i] + ach[1] * (U[1 + i] + Z1l[1 + i] + Z2l[1 + i])
        new[4 + i] = st[4 + i] + ach[2] * (U[4 + i] + Z1l[4 + i] + Z2l[4 + i])
    new[7] = st[7] + ach[3] * (U[7] + Z1l[7] + Z2l[7])
    return new


_NODE_IN_SPECS = [
    pl.BlockSpec((NB, 8 * D), lambda i: (i, 0)),
    pl.BlockSpec((NB, 8 * D), lambda i: (i, 0)),
    pl.BlockSpec((NB, 1), lambda i: (i, 0)),
    pl.BlockSpec((D, D), lambda i: (0, 0)),
    pl.BlockSpec((D, D), lambda i: (0, 0)),
    pl.BlockSpec((NUM_RBF, D), lambda i: (0, 0)),
    pl.BlockSpec((NUM_RBF, D), lambda i: (0, 0)),
    pl.BlockSpec((D, D), lambda i: (0, 0)),
    pl.BlockSpec((D, D), lambda i: (0, 0)),
    pl.BlockSpec((NUM_TYPES, 2 * D, 4 * D), lambda i: (0, 0, 0)),
    pl.BlockSpec((NUM_TYPES, 1, 4 * D), lambda i: (0, 0, 0)),
    pl.BlockSpec((NUM_TYPES, 4 * D, 4 * D), lambda i: (0, 0, 0)),
    pl.BlockSpec((NUM_TYPES, 1, 4 * D), lambda i: (0, 0, 0)),
]


def _node_msrc_body(st_ref, agg_ref, an_ref, uw_ref, vw_ref, z1w_ref,
                    z2w_ref, g1_ref, g2_ref, w1_ref, b1_ref, w2_ref, b2_ref,
                    pw1_ref, pb1_ref, pw2_ref, pb2_ref,
                    out_ref, a_ref, b_ref):
    new = _node_core(st_ref, agg_ref, an_ref, uw_ref, vw_ref, z1w_ref,
                     z2w_ref, g1_ref, g2_ref, w1_ref, b1_ref, w2_ref, b2_ref)
    out_ref[...] = jnp.concatenate(new, axis=1)
    st_rest = jnp.concatenate(new[1:], axis=1)
    _phi_tables(new[0], st_rest, pw1_ref, pb1_ref, pw2_ref, pb2_ref,
                a_ref, b_ref)


def _node_update_msrc(state, agg, an2, uw, vw, z1w, z2w, g1, g2, w1, b1, w2,
                      b2, pw1, pb1, pw2, pb2):
    return pl.pallas_call(
        _node_msrc_body,
        grid=(NBLK,),
        in_specs=_NODE_IN_SPECS + [
            pl.BlockSpec((D, 5 * D), lambda i: (0, 0)),
            pl.BlockSpec((1, 5 * D), lambda i: (0, 0)),
            pl.BlockSpec((5 * D, 5 * D), lambda i: (0, 0)),
            pl.BlockSpec((1, 5 * D), lambda i: (0, 0)),
        ],
        out_specs=[
            pl.BlockSpec((NB, 8 * D), lambda i: (i, 0)),
            pl.BlockSpec((NB, MA_W), lambda i: (i, 0)),
            pl.BlockSpec((NB, MB_W), lambda i: (i, 0)),
        ],
        out_shape=[
            jax.ShapeDtypeStruct((N, 8 * D), jnp.float32),
            jax.ShapeDtypeStruct((N, MA_W), jnp.float32),
            jax.ShapeDtypeStruct((N, MB_W), jnp.float32),
        ],
    )(state, agg, an2, uw, vw, z1w, z2w, g1, g2, w1, b1, w2, b2,
      pw1, pb1, pw2, pb2)


def _node_readout_body(st_ref, agg_ref, an_ref, uw_ref, vw_ref, z1w_ref,
                       z2w_ref, g1_ref, g2_ref, w1_ref, b1_ref, w2_ref,
                       b2_ref, co_ref, ngi_ref, out_ref, acc):
    i = pl.program_id(0)

    @pl.when(i == 0)
    def _init():
        acc[...] = jnp.zeros((G, D), jnp.float32)

    new = _node_core(st_ref, agg_ref, an_ref, uw_ref, vw_ref, z1w_ref,
                     z2w_ref, g1_ref, g2_ref, w1_ref, b1_ref, w2_ref, b2_ref)
    q = jnp.sum(new[0], axis=1, keepdims=True)
    cols = []
    for ax in range(3):
        m = jnp.sum(new[1 + ax], axis=1, keepdims=True)
        cols.append(m + q * co_ref[:, ax:ax + 1])
    mu_pad = jnp.concatenate(cols + [jnp.zeros((NB, D - 3), jnp.float32)], axis=1)
    ngi = ngi_ref[0]
    gcol = lax.broadcasted_iota(jnp.int32, (G, 1), 0)
    ohT = (gcol == ngi).astype(jnp.float32)
    acc[...] = acc[...] + jnp.dot(ohT, mu_pad, preferred_element_type=jnp.float32)

    @pl.when(i == NBLK - 1)
    def _fin():
        d3 = acc[:, 0:3]
        out_ref[...] = jnp.sqrt(jnp.sum(d3 * d3, axis=1, keepdims=True))


def _node_update_readout(state, agg, an2, uw, vw, z1w, z2w, g1, g2, w1, b1,
                         w2, b2, coords, ngi3):
    return pl.pallas_call(
        _node_readout_body,
        grid=(NBLK,),
        in_specs=_NODE_IN_SPECS + [
            pl.BlockSpec((NB, 3), lambda i: (i, 0)),
            pl.BlockSpec((1, 1, NB), lambda i: (i, 0, 0)),
        ],
        out_specs=pl.BlockSpec((G, 1), lambda i: (0, 0)),
        out_shape=jax.ShapeDtypeStruct((G, 1), jnp.float32),
        scratch_shapes=[pltpu.VMEM((G, D), jnp.float32)],
    )(state, agg, an2, uw, vw, z1w, z2w, g1, g2, w1, b1, w2, b2, coords, ngi3)


def kernel(num_nodes, num_graphs, atomic_numbers, edge_list, edge_lengths,
           edge_vectors, node_coordinates, node_graph_index, emb_scalar,
           emb_tri, rbf_W, rbf_b, phi_W1, phi_b1, phi_W2, phi_b2, U_W, V_W,
           Z1w, Z2w, gpZ1_W, gpZ2_W, upd_W1, upd_b1, upd_W2, upd_b2):
    an2 = atomic_numbers.reshape(N, 1)
    ngi3 = node_graph_index.reshape(NBLK, 1, NB)
    senders = edge_list[:, 0]
    receivers = edge_list[:, 1]
    el2 = edge_lengths.reshape(E, 1)

    pb1 = [phi_b1[rr].reshape(1, 5 * D) for rr in range(2)]
    pb2 = [phi_b2[rr].reshape(1, 5 * D) for rr in range(2)]
    ub1 = [upd_b1[rr].reshape(NUM_TYPES, 1, 4 * D) for rr in range(2)]
    ub2 = [upd_b2[rr].reshape(NUM_TYPES, 1, 4 * D) for rr in range(2)]

    state, ma, mb = _init_state(an2, emb_scalar, emb_tri,
                                phi_W1[0], pb1[0], phi_W2[0], pb2[0])
    rbts = [_rbt(el2, edge_vectors, rbf_W[rr], rbf_b[rr].reshape(1, 5 * D))
            for rr in range(2)]
    agg = _edge_aggregate(senders, receivers, ma, mb, rbts[0])
    state, ma, mb = _node_update_msrc(
        state, agg, an2, U_W[0], V_W[0], Z1w[0], Z2w[0],
        gpZ1_W[0], gpZ2_W[0], upd_W1[0], ub1[0], upd_W2[0], ub2[0],
        phi_W1[1], pb1[1], phi_W2[1], pb2[1])
    agg = _edge_aggregate(senders, receivers, ma, mb, rbts[1])
    return _node_update_readout(
        state, agg, an2, U_W[1], V_W[1], Z1w[1], Z2w[1],
        gpZ1_W[1], gpZ2_W[1], upd_W1[1], ub1[1], upd_W2[1], ub2[1],
        node_coordinates, ngi3)


# speedup vs baseline: 12.2311x; 1.0241x over previous
import functools
import math

import jax
import jax.numpy as jnp
from jax import lax
from jax.experimental import pallas as pl
from jax.experimental.pallas import tpu as pltpu
from jax.experimental.pallas import tpu_sc as plsc

N = 10000
E = 160000
D = 128
G = 64
NUM_RBF = 20
R_CUT = 5.0
NUM_TYPES = 5

_GP_IDX = ((0, 1, 2, 3, 4, 5, 6, 7), (1, 0, 4, 14, 2, 7, 11, 5), (2, 12, 0, 5, 9, 3, 7, 6), (3, 6, 13, 0, 7, 10, 1, 4), (4, 10, 1, 7, 8, 14, 5, 11), (5, 7, 11, 2, 6, 8, 12, 9), (6, 3, 7, 9, 13, 4, 8, 10), (7, 5, 6, 4, 11, 9, 10, 8))
_W_IDX = ((0, 1, 1, 1, 2, 2, 2, 3), (4, 5, 6, 6, 7, 8, 7, 9), (4, 6, 5, 6, 7, 7, 8, 9), (4, 6, 6, 5, 8, 7, 7, 9), (10, 11, 11, 12, 13, 14, 14, 15), (10, 12, 11, 11, 14, 13, 14, 15), (10, 11, 12, 11, 14, 14, 13, 15), (16, 17, 17, 17, 18, 18, 18, 19))

NB = 400
NBLK = N // NB
EB = 2000
RBT_W = 656
MSRC_W = 1536

NCHUNK = 23
CH = 448
NPAD = NCHUNK * CH
ACC_ROWS = 512
DUMMY = 511
EPT = E // 16
STRIP = 2000
NSTRIP = EPT // STRIP
K = 16
MA_W = 8 * D
MB_W = 4 * D


def _silu(x):
    return x / (1.0 + jnp.exp(-x))


def _phi_tables(s0, st_rest, w1_ref, b1_ref, w2_ref, b2_ref, a_ref, b_ref):
    h = _silu(jnp.dot(s0, w1_ref[...], preferred_element_type=jnp.float32)
              + b1_ref[...])
    phi = jnp.dot(h, w2_ref[...], preferred_element_type=jnp.float32) + b2_ref[...]
    a_ref[:, 0:D] = phi[:, 0:D]
    a_ref[:, D:8 * D] = st_rest
    b_ref[...] = phi[:, D:5 * D]


def _init_body(an_ref, es_ref, et_ref, w1_ref, b1_ref, w2_ref, b2_ref,
               out_ref, a_ref, b_ref):
    an = an_ref[...]
    tt = lax.broadcasted_iota(jnp.int32, (1, NUM_TYPES), 1)
    oh = (an == tt).astype(jnp.float32)
    s0 = jnp.dot(oh, es_ref[...], preferred_element_type=jnp.float32)
    s7 = jnp.dot(oh, et_ref[...], preferred_element_type=jnp.float32)
    zeros6 = jnp.zeros((NB, 6 * D), jnp.float32)
    out_ref[:, 0:D] = s0
    out_ref[:, D:7 * D] = zeros6
    out_ref[:, 7 * D:8 * D] = s7
    st_rest = jnp.concatenate([zeros6, s7], axis=1)
    _phi_tables(s0, st_rest, w1_ref, b1_ref, w2_ref, b2_ref, a_ref, b_ref)


def _init_state(an2, emb_s, emb_t, w1, b1, w2, b2):
    return pl.pallas_call(
        _init_body,
        grid=(NBLK,),
        in_specs=[
            pl.BlockSpec((NB, 1), lambda i: (i, 0)),
            pl.BlockSpec((NUM_TYPES, D), lambda i: (0, 0)),
            pl.BlockSpec((NUM_TYPES, D), lambda i: (0, 0)),
            pl.BlockSpec((D, 5 * D), lambda i: (0, 0)),
            pl.BlockSpec((1, 5 * D), lambda i: (0, 0)),
            pl.BlockSpec((5 * D, 5 * D), lambda i: (0, 0)),
            pl.BlockSpec((1, 5 * D), lambda i: (0, 0)),
        ],
        out_specs=[
            pl.BlockSpec((NB, 8 * D), lambda i: (i, 0)),
            pl.BlockSpec((NB, MA_W), lambda i: (i, 0)),
            pl.BlockSpec((NB, MB_W), lambda i: (i, 0)),
        ],
        out_shape=[
            jax.ShapeDtypeStruct((N, 8 * D), jnp.float32),
            jax.ShapeDtypeStruct((N, MA_W), jnp.float32),
            jax.ShapeDtypeStruct((N, MB_W), jnp.float32),
        ],
    )(an2, emb_s, emb_t, w1, b1, w2, b2)


def _rbt_body(el_ref, ev_ref, w_ref, b_ref, out_ref):
    el = el_ref[...]
    r = jnp.maximum(el, 1e-6)
    k = lax.broadcasted_iota(jnp.int32, (1, NUM_RBF), 1).astype(jnp.float32) + 1.0
    freqs = k * (math.pi / R_CUT)
    rbf = jnp.sin(r * freqs) / r
    cutoff = 0.5 * (jnp.cos((math.pi / R_CUT) * el) + 1.0)
    cutoff = cutoff * (el < R_CUT).astype(jnp.float32)
    out640 = (jnp.dot(rbf, w_ref[...], preferred_element_type=jnp.float32)
              + b_ref[...]) * cutoff
    out_ref[:, 0:5 * D] = out640
    out_ref[:, 5 * D:5 * D + 3] = ev_ref[...]
    out_ref[:, 5 * D + 3:RBT_W] = jnp.zeros((EB, RBT_W - 5 * D - 3), jnp.float32)


def _rbt(el2, ev, w, b2):
    return pl.pallas_call(
        _rbt_body,
        grid=(E // EB,),
        in_specs=[
            pl.BlockSpec((EB, 1), lambda i: (i, 0)),
            pl.BlockSpec((EB, 3), lambda i: (i, 0)),
            pl.BlockSpec((NUM_RBF, 5 * D), lambda i: (0, 0)),
            pl.BlockSpec((1, 5 * D), lambda i: (0, 0)),
        ],
        out_specs=pl.BlockSpec((EB, RBT_W), lambda i: (i, 0)),
        out_shape=jax.ShapeDtypeStruct((E, RBT_W), jnp.float32),
    )(el2, ev, w, b2)


def _edge_body(snd_hbm, rcv_hbm, ma_hbm, mb_hbm, rbt_hbm, agg_hbm,
               acc, sstrip, rstrip, leid, lsnd,
               ma0, mb0, rr0, rb0, ei0, si0, ri0,
               ma1, mb1, rr1, rb1, ei1, si1, ri1,
               sA0, sB0, sC0, sR0, sS0,
               sA1, sB1, sC1, sR1, sS1):
    core = lax.axis_index("c")
    sub = lax.axis_index("s")
    tile_ebase = sub * EPT
    lanes = lax.iota(jnp.int32, 16)
    slots = [
        (ma0, mb0, rr0, rb0, ei0, si0, ri0, sA0, sB0, sC0, sR0, sS0),
        (ma1, mb1, rr1, rb1, ei1, si1, ri1, sA1, sB1, sC1, sR1, sS1),
    ]

    def issue(b, bidx, cnt):
        ma, mb_, rr, rb, ei, si = slots[b][:6]
        sA, sB, sC, sR = slots[b][7:11]
        valid = (bidx * 16 + lanes) < cnt
        off = jnp.minimum(bidx * 16, EPT - 16)
        ei[...] = jnp.where(valid, leid[pl.ds(off, 16)], 0)
        si[...] = jnp.where(valid, lsnd[pl.ds(off, 16)], 0)
        pltpu.async_copy(ma_hbm.at[si], ma, sA)
        pltpu.async_copy(mb_hbm.at[si], mb_, sB)
        pltpu.async_copy(rbt_hbm.at[ei], rr, sC)
        pltpu.async_copy(rcv_hbm.at[ei], rb, sR)

    def wait_gathers(b):
        ma, mb_, rr, rb, ei, si = slots[b][:6]
        sA, sB, sC, sR = slots[b][7:11]
        pltpu.make_async_copy(ma_hbm.at[si], ma, sA).wait()
        pltpu.make_async_copy(mb_hbm.at[si], mb_, sB).wait()
        pltpu.make_async_copy(rbt_hbm.at[ei], rr, sC).wait()
        pltpu.make_async_copy(rcv_hbm.at[ei], rb, sR).wait()

    def compute_scatter(b, bidx, cnt, lo):
        ma, mb_, rr, rb, ei, si, ri = slots[b][:7]
        sS = slots[b][11]
        valid = (bidx * 16 + lanes) < cnt
        ri[...] = jnp.where(valid, rb[...] - lo, DUMMY)

        def edge_compute(e, _):
            evv = rr[e, pl.ds(5 * D, 16)]
            ev0 = evv[0]
            ev1 = evv[1]
            ev2 = evv[2]
            for j in range(8):
                o = j * 16
                rs = rr[e, pl.ds(o, 16)]
                rv_ = rr[e, pl.ds(D + o, 16)]
                rd_ = rr[e, pl.ds(2 * D + o, 16)]
                rb_ = rr[e, pl.ds(3 * D + o, 16)]
                rt_ = rr[e, pl.ds(4 * D + o, 16)]
                ps = ma[e, pl.ds(o, 16)]
                sv0 = ma[e, pl.ds(D + o, 16)]
                sv1 = ma[e, pl.ds(2 * D + o, 16)]
                sv2 = ma[e, pl.ds(3 * D + o, 16)]
                sb0 = ma[e, pl.ds(4 * D + o, 16)]
                sb1 = ma[e, pl.ds(5 * D + o, 16)]
                sb2 = ma[e, pl.ds(6 * D + o, 16)]
                st_ = ma[e, pl.ds(7 * D + o, 16)]
                g_v = mb_[e, pl.ds(o, 16)] * rv_
                g_d = mb_[e, pl.ds(D + o, 16)] * rd_
                g_b = mb_[e, pl.ds(2 * D + o, 16)] * rb_
                g_t = mb_[e, pl.ds(3 * D + o, 16)] * rt_
                ma[e, pl.ds(o, 16)] = ps * rs
                ma[e, pl.ds(D + o, 16)] = g_v * sv0 + g_d * ev0
                ma[e, pl.ds(2 * D + o, 16)] = g_v * sv1 + g_d * ev1
                ma[e, pl.ds(3 * D + o, 16)] = g_v * sv2 + g_d * ev2
                ma[e, pl.ds(4 * D + o, 16)] = g_b * sb0
                ma[e, pl.ds(5 * D + o, 16)] = g_b * sb1
                ma[e, pl.ds(6 * D + o, 16)] = g_b * sb2
                ma[e, pl.ds(7 * D + o, 16)] = g_t * st_
            return 0
        lax.fori_loop(0, K, edge_compute, 0)
        pltpu.async_copy(ma, acc.at[ri], sS, add=True)

    def wait_scatter(b):
        ma, ri, sS = slots[b][0], slots[b][6], slots[b][11]
        pltpu.make_async_copy(ma, acc.at[ri], sS).wait()

    def scan_edges(lo, hi):
        def strip_body(s, cnt):
            sbase = tile_ebase + s * STRIP
            pltpu.sync_copy(snd_hbm.at[pl.ds(sbase, STRIP)], sstrip)
            pltpu.sync_copy(rcv_hbm.at[pl.ds(sbase, STRIP)], rstrip)

            def vbody(j, cnt):
                rv = rstrip[pl.ds(j * 16, 16)]
                sv = sstrip[pl.ds(j * 16, 16)]
                m = (rv >= lo) & (rv < hi)
                mi = m.astype(jnp.int32)
                pos = cnt + plsc.cumsum(mi) - 1
                eidv = sbase + j * 16 + lanes
                plsc.store_scatter(leid, [pos], eidv, mask=m)
                plsc.store_scatter(lsnd, [pos], sv, mask=m)
                return cnt + jnp.sum(mi)
            return lax.fori_loop(0, STRIP // 16, vbody, cnt)
        return lax.fori_loop(0, NSTRIP, strip_body, jnp.int32(0))

    def process(cnt, lo):
        nb = (cnt + 15) // 16
        npair = (nb + 1) // 2
        issue(0, 0, cnt)
        issue(1, 1, cnt)

        def pair_body(p, _):
            for b in range(2):
                bidx = 2 * p + b
                wait_gathers(b)
                compute_scatter(b, bidx, cnt, lo)
                wait_scatter(b)
                issue(b, bidx + 2, cnt)
            return 0
        lax.fori_loop(0, npair, pair_body, 0)
        for b in range(2):
            wait_gathers(b)

    def chunk_body(chunk, _):
        lo = chunk * CH

        @pl.when(core == (chunk % 2))
        def _process():
            def zb(e, _):
                for j in range(64):
                    ma0[e, pl.ds(j * 16, 16)] = jnp.zeros((16,), jnp.float32)
                return 0
            lax.fori_loop(0, K, zb, 0)
            pltpu.sync_copy(ma0, acc.at[pl.ds(sub * 32, 16)])
            pltpu.sync_copy(ma0, acc.at[pl.ds(sub * 32 + 16, 16)])
            plsc.subcore_barrier()
            cnt = scan_edges(lo, lo + CH)
            process(cnt, lo)
            plsc.subcore_barrier()

            @pl.when(sub < 8)
            def _writeback():
                pltpu.sync_copy(acc.at[pl.ds(sub * (CH // 8), CH // 8)],
                                agg_hbm.at[pl.ds(lo + sub * (CH // 8), CH // 8)])
            plsc.subcore_barrier()
        return 0

    lax.fori_loop(0, NCHUNK, chunk_body, 0)


def _edge_aggregate(senders, receivers, msrc_a, msrc_b, rbt):
    mesh = plsc.VectorSubcoreMesh(core_axis_name="c", subcore_axis_name="s")
    vm = pltpu.VMEM
    slot = [vm((K, MA_W), jnp.float32), vm((K, MB_W), jnp.float32),
            vm((K, RBT_W), jnp.float32), vm((16,), jnp.int32),
            vm((16,), jnp.int32), vm((16,), jnp.int32), vm((16,), jnp.int32)]
    f = pl.kernel(
        _edge_body,
        mesh=mesh,
        compiler_params=pltpu.CompilerParams(use_tc_tiling_on_sc=False,
                                             needs_layout_passes=False),
        out_type=jax.ShapeDtypeStruct((NPAD, 8 * D), jnp.float32),
        scratch_types=(
            [pltpu.VMEM_SHARED((ACC_ROWS, 8 * D), jnp.float32),
             vm((STRIP,), jnp.int32), vm((STRIP,), jnp.int32),
             vm((EPT,), jnp.int32), vm((EPT,), jnp.int32)]
            + slot + slot
            + [pltpu.SemaphoreType.DMA] * 10
        ),
    )
    return f(senders, receivers, msrc_a, msrc_b, rbt)


def _node_core(st_ref, agg_ref, an_ref, uw_ref, vw_ref, z1w_ref, z2w_ref,
               g1_ref, g2_ref, w1_ref, b1_ref, w2_ref, b2_ref):
    st = [st_ref[:, c * D:(c + 1) * D] + agg_ref[:, c * D:(c + 1) * D]
          for c in range(8)]
    uw = uw_ref[...]
    vw = vw_ref[...]
    U = [jnp.dot(st[c], uw, preferred_element_type=jnp.float32) for c in range(8)]
    V = [jnp.dot(st[c], vw, preferred_element_type=jnp.float32) for c in range(8)]

    def wmp(A, B, w_ref):
        out = [None] * 8
        for i in range(8):
            for j in range(8):
                gp = _GP_IDX[i][j]
                w = w_ref[_W_IDX[i][j]:_W_IDX[i][j] + 1, :]
                term = A[i] * B[j] * w
                c = gp if gp < 8 else gp - 8
                if out[c] is None:
                    out[c] = term if gp < 8 else -term
                else:
                    out[c] = out[c] + term if gp < 8 else out[c] - term
        return out

    Z1 = wmp(U, V, z1w_ref)
    g1 = g1_ref[...]
    Z1l = [jnp.dot(Z1[c], g1, preferred_element_type=jnp.float32) for c in range(8)]
    Z2 = wmp(U, Z1l, z2w_ref)
    g2 = g2_ref[...]
    Z2l = [jnp.dot(Z2[c], g2, preferred_element_type=jnp.float32) for c in range(8)]

    v_norm = jnp.sqrt(V[1] * V[1] + V[2] * V[2] + V[3] * V[3])
    upd_in = jnp.concatenate([st[0], v_norm], axis=1)
    an = an_ref[...]
    a = jnp.zeros((NB, 4 * D), jnp.float32)
    for t in range(NUM_TYPES):
        h1 = _silu(jnp.dot(upd_in, w1_ref[t], preferred_element_type=jnp.float32)
                   + b1_ref[t])
        out_t = jnp.dot(h1, w2_ref[t], preferred_element_type=jnp.float32) + b2_ref[t]
        a = jnp.where(an == t, out_t, a)
    ach = [a[:, q * D:(q + 1) * D] for q in range(4)]
    new = [None] * 8
    new[0] = st[0] + ach[0] * (U[0] + Z1l[0] + Z2l[0])
    for i in range(3):
        new[1 + i] = st[1 + i] + ach[1] * (U[1 + i] + Z1l[1 + i] + Z2l[1 + i])
        new[4 + i] = st[4 + i] + ach[2] * (U[4 + i] + Z1l[4 + i] + Z2l[4 + i])
    new[7] = st[7] + ach[3] * (U[7] + Z1l[7] + Z2l[7])
    return new


_NODE_IN_SPECS = [
    pl.BlockSpec((NB, 8 * D), lambda i: (i, 0)),
    pl.BlockSpec((NB, 8 * D), lambda i: (i, 0)),
    pl.BlockSpec((NB, 1), lambda i: (i, 0)),
    pl.BlockSpec((D, D), lambda i: (0, 0)),
    pl.BlockSpec((D, D), lambda i: (0, 0)),
    pl.BlockSpec((NUM_RBF, D), lambda i: (0, 0)),
    pl.BlockSpec((NUM_RBF, D), lambda i: (0, 0)),
    pl.BlockSpec((D, D), lambda i: (0, 0)),
    pl.BlockSpec((D, D), lambda i: (0, 0)),
    pl.BlockSpec((NUM_TYPES, 2 * D, 4 * D), lambda i: (0, 0, 0)),
    pl.BlockSpec((NUM_TYPES, 1, 4 * D), lambda i: (0, 0, 0)),
    pl.BlockSpec((NUM_TYPES, 4 * D, 4 * D), lambda i: (0, 0, 0)),
    pl.BlockSpec((NUM_TYPES, 1, 4 * D), lambda i: (0, 0, 0)),
]


def _node_msrc_body(st_ref, agg_ref, an_ref, uw_ref, vw_ref, z1w_ref,
                    z2w_ref, g1_ref, g2_ref, w1_ref, b1_ref, w2_ref, b2_ref,
                    pw1_ref, pb1_ref, pw2_ref, pb2_ref,
                    out_ref, a_ref, b_ref):
    new = _node_core(st_ref, agg_ref, an_ref, uw_ref, vw_ref, z1w_ref,
                     z2w_ref, g1_ref, g2_ref, w1_ref, b1_ref, w2_ref, b2_ref)
    out_ref[...] = jnp.concatenate(new, axis=1)
    st_rest = jnp.concatenate(new[1:], axis=1)
    _phi_tables(new[0], st_rest, pw1_ref, pb1_ref, pw2_ref, pb2_ref,
                a_ref, b_ref)


def _node_update_msrc(state, agg, an2, uw, vw, z1w, z2w, g1, g2, w1, b1, w2,
                      b2, pw1, pb1, pw2, pb2):
    return pl.pallas_call(
        _node_msrc_body,
        grid=(NBLK,),
        in_specs=_NODE_IN_SPECS + [
            pl.BlockSpec((D, 5 * D), lambda i: (0, 0)),
            pl.BlockSpec((1, 5 * D), lambda i: (0, 0)),
            pl.BlockSpec((5 * D, 5 * D), lambda i: (0, 0)),
            pl.BlockSpec((1, 5 * D), lambda i: (0, 0)),
        ],
        out_specs=[
            pl.BlockSpec((NB, 8 * D), lambda i: (i, 0)),
            pl.BlockSpec((NB, MA_W), lambda i: (i, 0)),
            pl.BlockSpec((NB, MB_W), lambda i: (i, 0)),
        ],
        out_shape=[
            jax.ShapeDtypeStruct((N, 8 * D), jnp.float32),
            jax.ShapeDtypeStruct((N, MA_W), jnp.float32),
            jax.ShapeDtypeStruct((N, MB_W), jnp.float32),
        ],
    )(state, agg, an2, uw, vw, z1w, z2w, g1, g2, w1, b1, w2, b2,
      pw1, pb1, pw2, pb2)


def _node_readout_body(st_ref, agg_ref, an_ref, uw_ref, vw_ref, z1w_ref,
                       z2w_ref, g1_ref, g2_ref, w1_ref, b1_ref, w2_ref,
                       b2_ref, co_ref, ngi_ref, out_ref, acc):
    i = pl.program_id(0)

    @pl.when(i == 0)
    def _init():
        acc[...] = jnp.zeros((G, D), jnp.float32)

    new = _node_core(st_ref, agg_ref, an_ref, uw_ref, vw_ref, z1w_ref,
                     z2w_ref, g1_ref, g2_ref, w1_ref, b1_ref, w2_ref, b2_ref)
    q = jnp.sum(new[0], axis=1, keepdims=True)
    cols = []
    for ax in range(3):
        m = jnp.sum(new[1 + ax], axis=1, keepdims=True)
        cols.append(m + q * co_ref[:, ax:ax + 1])
    mu_pad = jnp.concatenate(cols + [jnp.zeros((NB, D - 3), jnp.float32)], axis=1)
    ngi = ngi_ref[0]
    gcol = lax.broadcasted_iota(jnp.int32, (G, 1), 0)
    ohT = (gcol == ngi).astype(jnp.float32)
    acc[...] = acc[...] + jnp.dot(ohT, mu_pad, preferred_element_type=jnp.float32)

    @pl.when(i == NBLK - 1)
    def _fin():
        d3 = acc[:, 0:3]
        out_ref[...] = jnp.sqrt(jnp.sum(d3 * d3, axis=1, keepdims=True))


def _node_update_readout(state, agg, an2, uw, vw, z1w, z2w, g1, g2, w1, b1,
                         w2, b2, coords, ngi3):
    return pl.pallas_call(
        _node_readout_body,
        grid=(NBLK,),
        in_specs=_NODE_IN_SPECS + [
            pl.BlockSpec((NB, 3), lambda i: (i, 0)),
            pl.BlockSpec((1, 1, NB), lambda i: (i, 0, 0)),
        ],
        out_specs=pl.BlockSpec((G, 1), lambda i: (0, 0)),
        out_shape=jax.ShapeDtypeStruct((G, 1), jnp.float32),
        scratch_shapes=[pltpu.VMEM((G, D), jnp.float32)],
    )(state, agg, an2, uw, vw, z1w, z2w, g1, g2, w1, b1, w2, b2, coords, ngi3)


def kernel(num_nodes, num_graphs, atomic_numbers, edge_list, edge_lengths,
           edge_vectors, node_coordinates, node_graph_index, emb_scalar,
           emb_tri, rbf_W, rbf_b, phi_W1, phi_b1, phi_W2, phi_b2, U_W, V_W,
           Z1w, Z2w, gpZ1_W, gpZ2_W, upd_W1, upd_b1, upd_W2, upd_b2):
    an2 = atomic_numbers.reshape(N, 1)
    ngi3 = node_graph_index.reshape(NBLK, 1, NB)
    senders = edge_list[:, 0]
    receivers = edge_list[:, 1]
    el2 = edge_lengths.reshape(E, 1)

    pb1 = [phi_b1[rr].reshape(1, 5 * D) for rr in range(2)]
    pb2 = [phi_b2[rr].reshape(1, 5 * D) for rr in range(2)]
    ub1 = [upd_b1[rr].reshape(NUM_TYPES, 1, 4 * D) for rr in range(2)]
    ub2 = [upd_b2[rr].reshape(NUM_TYPES, 1, 4 * D) for rr in range(2)]

    state, ma, mb = _init_state(an2, emb_scalar, emb_tri,
                                phi_W1[0], pb1[0], phi_W2[0], pb2[0])
    rbts = [_rbt(el2, edge_vectors, rbf_W[rr], rbf_b[rr].reshape(1, 5 * D))
            for rr in range(2)]
    agg = _edge_aggregate(senders, receivers, ma, mb, rbts[0])
    state, ma, mb = _node_update_msrc(
        state, agg, an2, U_W[0], V_W[0], Z1w[0], Z2w[0],
        gpZ1_W[0], gpZ2_W[0], upd_W1[0], ub1[0], upd_W2[0], ub2[0],
        phi_W1[1], pb1[1], phi_W2[1], pb2[1])
    agg = _edge_aggregate(senders, receivers, ma, mb, rbts[1])
    return _node_update_readout(
        state, agg, an2, U_W[1], V_W[1], Z1w[1], Z2w[1],
        gpZ1_W[1], gpZ2_W[1], upd_W1[1], ub1[1], upd_W2[1], ub2[1],
        node_coordinates, ngi3)
